# Initial kernel scaffold; baseline (speedup 1.0000x reference)
import jax
import jax.numpy as jnp
from jax.experimental import pallas as pl


def kernel(x, edge_index, params):
    raise NotImplementedError("write your pallas kernel here")


# speedup vs baseline: 2.0101x; 2.0101x over previous
import functools

import jax
import jax.numpy as jnp
from jax import lax
from jax.experimental import pallas as pl
from jax.experimental.pallas import tpu as pltpu

N_NODES = 10000
N_EDGES = 320000
HID = 128
GEN_EPS = 1e-7
LOG_EPS = 0.01
_PREC = lax.Precision.HIGHEST

ROW_BLOCK = 1000
N_ROW_BLOCKS = N_NODES // ROW_BLOCK

_INTERPRET = False


def _ln(y, g, b):
    mu = jnp.mean(y, axis=-1, keepdims=True)
    var = jnp.mean((y - mu) ** 2, axis=-1, keepdims=True)
    return (y - mu) / jnp.sqrt(var + 1e-5) * g + b


def _dot(a, b):
    return jnp.dot(a, b, preferred_element_type=jnp.float32, precision=_PREC)


def _fc_body(x_ref, w_ref, b_ref, o_ref):
    o_ref[...] = jax.nn.relu(_dot(x_ref[...], w_ref[...]) + b_ref[...])


def _fc(x, w, b):
    in_dim = x.shape[1]
    return pl.pallas_call(
        _fc_body,
        grid=(N_ROW_BLOCKS,),
        in_specs=[
            pl.BlockSpec((ROW_BLOCK, in_dim), lambda i: (i, 0)),
            pl.BlockSpec((in_dim, HID), lambda i: (0, 0)),
            pl.BlockSpec((1, HID), lambda i: (0, 0)),
        ],
        out_specs=pl.BlockSpec((ROW_BLOCK, HID), lambda i: (i, 0)),
        out_shape=jax.ShapeDtypeStruct((N_NODES, HID), jnp.float32),
        interpret=_INTERPRET,
    )(x, w, b.reshape(1, HID))


def _msgtab_body(t_ref, h_ref, o_ref):
    m = jax.nn.relu(h_ref[...]) + GEN_EPS
    e = jnp.exp(t_ref[0, 0] * m)
    em = e * m
    o_ref[...] = jnp.concatenate(
        [em[:, :64], e[:, :64], em[:, 64:], e[:, 64:]], axis=1)


def _msgtab(h, t):
    return pl.pallas_call(
        _msgtab_body,
        grid=(N_ROW_BLOCKS,),
        in_specs=[
            pl.BlockSpec((1, 1), lambda i: (0, 0)),
            pl.BlockSpec((ROW_BLOCK, HID), lambda i: (i, 0)),
        ],
        out_specs=pl.BlockSpec((ROW_BLOCK, 2 * HID), lambda i: (i, 0)),
        out_shape=jax.ShapeDtypeStruct((N_NODES, 2 * HID), jnp.float32),
        interpret=_INTERPRET,
    )(t.reshape(1, 1), h)


def _edge_pass(tab, src, dst):
    t2 = tab.reshape(2 * N_NODES, HID)
    acc0 = jax.ops.segment_sum(t2[src * 2], dst, num_segments=N_NODES)
    acc1 = jax.ops.segment_sum(t2[src * 2 + 1], dst, num_segments=N_NODES)
    return jnp.stack([acc0, acc1])


def _post_body(h_ref, num_ref, den_ref, w1_ref, b1_ref, g1_ref, bb1_ref,
               w2_ref, b2_ref, g2_ref, bb2_ref, o_ref, *, first):
    aggr = num_ref[...] / (den_ref[...] + 1e-16)
    out = h_ref[...] + aggr
    y = _dot(out, w1_ref[...]) + b1_ref[...]
    y = _ln(y, g1_ref[...], bb1_ref[...])
    y = jax.nn.relu(y)
    y = _dot(y, w2_ref[...]) + b2_ref[...]
    if first:
        o_ref[...] = y
    else:
        y = _ln(y, g2_ref[...], bb2_ref[...])
        o_ref[...] = h_ref[...] + jax.nn.relu(y)


def _post(h, numer, denom, lp, first):
    h2 = 2 * HID
    return pl.pallas_call(
        functools.partial(_post_body, first=first),
        grid=(N_ROW_BLOCKS,),
        in_specs=[
            pl.BlockSpec((ROW_BLOCK, HID), lambda i: (i, 0)),
            pl.BlockSpec((ROW_BLOCK, HID), lambda i: (i, 0)),
            pl.BlockSpec((ROW_BLOCK, HID), lambda i: (i, 0)),
            pl.BlockSpec((HID, h2), lambda i: (0, 0)),
            pl.BlockSpec((1, h2), lambda i: (0, 0)),
            pl.BlockSpec((1, h2), lambda i: (0, 0)),
            pl.BlockSpec((1, h2), lambda i: (0, 0)),
            pl.BlockSpec((h2, HID), lambda i: (0, 0)),
            pl.BlockSpec((1, HID), lambda i: (0, 0)),
            pl.BlockSpec((1, HID), lambda i: (0, 0)),
            pl.BlockSpec((1, HID), lambda i: (0, 0)),
        ],
        out_specs=pl.BlockSpec((ROW_BLOCK, HID), lambda i: (i, 0)),
        out_shape=jax.ShapeDtypeStruct((N_NODES, HID), jnp.float32),
        interpret=_INTERPRET,
    )(h, numer, denom,
      lp["w1"], lp["b1"].reshape(1, h2), lp["ln1_g"].reshape(1, h2),
      lp["ln1_b"].reshape(1, h2), lp["w2"], lp["b2"].reshape(1, HID),
      lp["ln_g"].reshape(1, HID), lp["ln_b"].reshape(1, HID))


def _head_body(h0_ref, h1_ref, h2_ref, h3_ref, phiw_ref, phib_ref,
               aw_ref, ab_ref, bw_ref, bb_ref, cw_ref, cb_ref, vw_ref,
               hw_ref, hb_ref, o_ref, se_ref, swa_ref, sp_ref, spp_ref):
    i = pl.program_id(0)

    @pl.when(i == 0)
    def _():
        se_ref[...] = jnp.zeros_like(se_ref)
        swa_ref[...] = jnp.zeros_like(swa_ref)
        sp_ref[...] = jnp.zeros_like(sp_ref)
        spp_ref[...] = jnp.zeros_like(spp_ref)

    xcat = jnp.concatenate(
        [h0_ref[...], h1_ref[...], h2_ref[...], h3_ref[...]], axis=1)
    hp = jax.nn.relu(_dot(xcat, phiw_ref[...]) + phib_ref[...])
    a = jnp.tanh(_dot(hp, aw_ref[...]) + ab_ref[...])
    g = jax.nn.sigmoid(_dot(hp, bw_ref[...]) + bb_ref[...])
    logit = _dot(a * g, cw_ref[...]) + cb_ref[...]
    e = jnp.exp(logit)
    proj = _dot(hp, vw_ref[...])
    se_ref[...] += jnp.sum(e).reshape(1, 1)
    swa_ref[...] += jnp.sum(e * hp, axis=0, keepdims=True)
    sp_ref[...] += jnp.sum(e * proj, axis=0, keepdims=True)
    spp_ref[...] += jnp.sum(e * proj * proj, axis=0, keepdims=True)

    @pl.when(i == N_ROW_BLOCKS - 1)
    def _():
        se = se_ref[...]
        wavg = swa_ref[...] / se
        mean = sp_ref[...] / se
        var = spp_ref[...] / se - mean * mean
        vp = jnp.log(var + LOG_EPS)
        merged = jnp.concatenate([wavg, vp], axis=1)
        o_ref[...] = _dot(merged, hw_ref[...]) + hb_ref[...]


def _head(hs, p):
    cat = 4 * HID
    nvp = p["var_w"].shape[1]
    body = pl.pallas_call(
        _head_body,
        grid=(N_ROW_BLOCKS,),
        in_specs=[pl.BlockSpec((ROW_BLOCK, HID), lambda i: (i, 0))] * 4 + [
            pl.BlockSpec((cat, cat), lambda i: (0, 0)),
            pl.BlockSpec((1, cat), lambda i: (0, 0)),
            pl.BlockSpec((cat, cat), lambda i: (0, 0)),
            pl.BlockSpec((1, cat), lambda i: (0, 0)),
            pl.BlockSpec((cat, cat), lambda i: (0, 0)),
            pl.BlockSpec((1, cat), lambda i: (0, 0)),
            pl.BlockSpec((cat, 1), lambda i: (0, 0)),
            pl.BlockSpec((1, 1), lambda i: (0, 0)),
            pl.BlockSpec((cat, nvp), lambda i: (0, 0)),
            pl.BlockSpec((cat + nvp, 4), lambda i: (0, 0)),
            pl.BlockSpec((1, 4), lambda i: (0, 0)),
        ],
        out_specs=pl.BlockSpec((1, 4), lambda i: (0, 0)),
        out_shape=jax.ShapeDtypeStruct((1, 4), jnp.float32),
        scratch_shapes=[
            pltpu.VMEM((1, 1), jnp.float32),
            pltpu.VMEM((1, cat), jnp.float32),
            pltpu.VMEM((1, nvp), jnp.float32),
            pltpu.VMEM((1, nvp), jnp.float32),
        ],
        interpret=_INTERPRET,
    )
    return body(hs[0], hs[1], hs[2], hs[3],
                p["phi_w"], p["phi_b"].reshape(1, cat),
                p["attn_a_w"], p["attn_a_b"].reshape(1, cat),
                p["attn_b_w"], p["attn_b_b"].reshape(1, cat),
                p["attn_c_w"], p["attn_c_b"].reshape(1, 1),
                p["var_w"], p["head_w"], p["head_b"].reshape(1, 4))


def kernel(x, edge_index, params):
    p = params
    src = edge_index[0].astype(jnp.int32)
    dst = edge_index[1].astype(jnp.int32)
    h = _fc(x, p["fc_w"], p["fc_b"])
    hs = [h]
    for i, lp in enumerate(p["layers"]):
        tab = _msgtab(h, lp["t"])
        acc = _edge_pass(tab, src, dst)
        numer = jnp.concatenate([acc[0, :, :64], acc[1, :, :64]], axis=1)
        denom = jnp.concatenate([acc[0, :, 64:], acc[1, :, 64:]], axis=1)
        h = _post(h, numer, denom, lp, first=(i == 0))
        hs.append(h)
    return _head(hs, p)


# speedup vs baseline: 6.0282x; 2.9989x over previous
import functools

import jax
import jax.numpy as jnp
from jax import lax
from jax.experimental import pallas as pl
from jax.experimental.pallas import tpu as pltpu
from jax.experimental.pallas import tpu_sc as plsc

N_NODES = 10000
N_EDGES = 320000
HID = 128
GEN_EPS = 1e-7
LOG_EPS = 0.01
_PREC = lax.Precision.HIGHEST

ROW_BLOCK = 1000
N_ROW_BLOCKS = N_NODES // ROW_BLOCK

_INTERPRET = False


def _ln(y, g, b):
    mu = jnp.mean(y, axis=-1, keepdims=True)
    var = jnp.mean((y - mu) ** 2, axis=-1, keepdims=True)
    return (y - mu) / jnp.sqrt(var + 1e-5) * g + b


def _dot(a, b):
    return jnp.dot(a, b, preferred_element_type=jnp.float32, precision=_PREC)


def _fc_body(x_ref, w_ref, b_ref, o_ref):
    o_ref[...] = jax.nn.relu(_dot(x_ref[...], w_ref[...]) + b_ref[...])


def _fc(x, w, b):
    in_dim = x.shape[1]
    return pl.pallas_call(
        _fc_body,
        grid=(N_ROW_BLOCKS,),
        in_specs=[
            pl.BlockSpec((ROW_BLOCK, in_dim), lambda i: (i, 0)),
            pl.BlockSpec((in_dim, HID), lambda i: (0, 0)),
            pl.BlockSpec((1, HID), lambda i: (0, 0)),
        ],
        out_specs=pl.BlockSpec((ROW_BLOCK, HID), lambda i: (i, 0)),
        out_shape=jax.ShapeDtypeStruct((N_NODES, HID), jnp.float32),
        interpret=_INTERPRET,
    )(x, w, b.reshape(1, HID))


def _msgtab_body(t_ref, h_ref, o_ref):
    m = jax.nn.relu(h_ref[...]) + GEN_EPS
    e = jnp.exp(t_ref[0, 0] * m)
    em = e * m
    o_ref[...] = jnp.concatenate(
        [em[:, :64], e[:, :64], em[:, 64:], e[:, 64:]], axis=1)


def _msgtab(h, t):
    return pl.pallas_call(
        _msgtab_body,
        grid=(N_ROW_BLOCKS,),
        in_specs=[
            pl.BlockSpec((1, 1), lambda i: (0, 0)),
            pl.BlockSpec((ROW_BLOCK, HID), lambda i: (i, 0)),
        ],
        out_specs=pl.BlockSpec((ROW_BLOCK, 2 * HID), lambda i: (i, 0)),
        out_shape=jax.ShapeDtypeStruct((N_NODES, 2 * HID), jnp.float32),
        interpret=_INTERPRET,
    )(t.reshape(1, 1), h)


_SC_SUBCORES = 16
_EDGES_PER_SUB = N_EDGES // _SC_SUBCORES
_W = 80
_NWIN = _EDGES_PER_SUB // _W
_RS = 624
_RS_LAST = N_NODES - 15 * _RS


def _sc_edge_body(tab_hbm, src_hbm, dst_hbm, zeros_hbm, out_hbm,
                  src_v, dst_v, gidx_v, rows_v, acc_sh, sem):
    c = lax.axis_index("c")
    s = lax.axis_index("s")
    cv = jnp.broadcast_to(c, (16,)).astype(jnp.int32)
    r0 = s * _RS

    @pl.when(s < 15)
    def _():
        pltpu.sync_copy(zeros_hbm.at[pl.ds(r0, _RS)],
                        acc_sh.at[pl.ds(r0, _RS)])

    @pl.when(s == 15)
    def _():
        pltpu.sync_copy(zeros_hbm.at[pl.ds(15 * _RS, _RS_LAST)],
                        acc_sh.at[pl.ds(15 * _RS, _RS_LAST)])

    plsc.subcore_barrier()
    base = s * _EDGES_PER_SUB

    @pl.loop(0, _NWIN)
    def _win(k):
        off = base + k * _W
        pltpu.sync_copy(src_hbm.at[pl.ds(off, _W)], src_v)
        pltpu.sync_copy(dst_hbm.at[pl.ds(off, _W)], dst_v)

        @pl.loop(0, _W, step=16)
        def _idx(j):
            gidx_v[pl.ds(j, 16)] = src_v[pl.ds(j, 16)] * 2 + cv

        pltpu.async_copy(tab_hbm.at[gidx_v], rows_v, sem).wait()
        pltpu.sync_copy(rows_v, acc_sh.at[dst_v], add=True)

    plsc.subcore_barrier()

    @pl.when(s < 15)
    def _():
        pltpu.sync_copy(acc_sh.at[pl.ds(r0, _RS)],
                        out_hbm.at[c, pl.ds(r0, _RS)])

    @pl.when(s == 15)
    def _():
        pltpu.sync_copy(acc_sh.at[pl.ds(15 * _RS, _RS_LAST)],
                        out_hbm.at[c, pl.ds(15 * _RS, _RS_LAST)])


def _edge_pass(tab, src, dst):
    t2 = tab.reshape(2 * N_NODES, HID)
    zeros = jnp.zeros((N_NODES, HID), jnp.float32)
    mesh = plsc.VectorSubcoreMesh(core_axis_name="c", subcore_axis_name="s")
    f = pl.kernel(
        _sc_edge_body,
        out_type=jax.ShapeDtypeStruct((2, N_NODES, HID), jnp.float32),
        mesh=mesh,
        scratch_types=[
            pltpu.VMEM((_W,), jnp.int32),
            pltpu.VMEM((_W,), jnp.int32),
            pltpu.VMEM((_W,), jnp.int32),
            pltpu.VMEM((_W, HID), jnp.float32),
            pltpu.VMEM_SHARED((N_NODES, HID), jnp.float32),
            pltpu.SemaphoreType.DMA,
        ],
    )
    return f(t2, src, dst, zeros)


def _post_body(h_ref, num_ref, den_ref, w1_ref, b1_ref, g1_ref, bb1_ref,
               w2_ref, b2_ref, g2_ref, bb2_ref, o_ref, *, first):
    aggr = num_ref[...] / (den_ref[...] + 1e-16)
    out = h_ref[...] + aggr
    y = _dot(out, w1_ref[...]) + b1_ref[...]
    y = _ln(y, g1_ref[...], bb1_ref[...])
    y = jax.nn.relu(y)
    y = _dot(y, w2_ref[...]) + b2_ref[...]
    if first:
        o_ref[...] = y
    else:
        y = _ln(y, g2_ref[...], bb2_ref[...])
        o_ref[...] = h_ref[...] + jax.nn.relu(y)


def _post(h, numer, denom, lp, first):
    h2 = 2 * HID
    return pl.pallas_call(
        functools.partial(_post_body, first=first),
        grid=(N_ROW_BLOCKS,),
        in_specs=[
            pl.BlockSpec((ROW_BLOCK, HID), lambda i: (i, 0)),
            pl.BlockSpec((ROW_BLOCK, HID), lambda i: (i, 0)),
            pl.BlockSpec((ROW_BLOCK, HID), lambda i: (i, 0)),
            pl.BlockSpec((HID, h2), lambda i: (0, 0)),
            pl.BlockSpec((1, h2), lambda i: (0, 0)),
            pl.BlockSpec((1, h2), lambda i: (0, 0)),
            pl.BlockSpec((1, h2), lambda i: (0, 0)),
            pl.BlockSpec((h2, HID), lambda i: (0, 0)),
            pl.BlockSpec((1, HID), lambda i: (0, 0)),
            pl.BlockSpec((1, HID), lambda i: (0, 0)),
            pl.BlockSpec((1, HID), lambda i: (0, 0)),
        ],
        out_specs=pl.BlockSpec((ROW_BLOCK, HID), lambda i: (i, 0)),
        out_shape=jax.ShapeDtypeStruct((N_NODES, HID), jnp.float32),
        interpret=_INTERPRET,
    )(h, numer, denom,
      lp["w1"], lp["b1"].reshape(1, h2), lp["ln1_g"].reshape(1, h2),
      lp["ln1_b"].reshape(1, h2), lp["w2"], lp["b2"].reshape(1, HID),
      lp["ln_g"].reshape(1, HID), lp["ln_b"].reshape(1, HID))


def _head_body(h0_ref, h1_ref, h2_ref, h3_ref, phiw_ref, phib_ref,
               aw_ref, ab_ref, bw_ref, bb_ref, cw_ref, cb_ref, vw_ref,
               hw_ref, hb_ref, o_ref, se_ref, swa_ref, sp_ref, spp_ref):
    i = pl.program_id(0)

    @pl.when(i == 0)
    def _():
        se_ref[...] = jnp.zeros_like(se_ref)
        swa_ref[...] = jnp.zeros_like(swa_ref)
        sp_ref[...] = jnp.zeros_like(sp_ref)
        spp_ref[...] = jnp.zeros_like(spp_ref)

    xcat = jnp.concatenate(
        [h0_ref[...], h1_ref[...], h2_ref[...], h3_ref[...]], axis=1)
    hp = jax.nn.relu(_dot(xcat, phiw_ref[...]) + phib_ref[...])
    a = jnp.tanh(_dot(hp, aw_ref[...]) + ab_ref[...])
    g = jax.nn.sigmoid(_dot(hp, bw_ref[...]) + bb_ref[...])
    logit = _dot(a * g, cw_ref[...]) + cb_ref[...]
    e = jnp.exp(logit)
    proj = _dot(hp, vw_ref[...])
    se_ref[...] += jnp.sum(e).reshape(1, 1)
    swa_ref[...] += jnp.sum(e * hp, axis=0, keepdims=True)
    sp_ref[...] += jnp.sum(e * proj, axis=0, keepdims=True)
    spp_ref[...] += jnp.sum(e * proj * proj, axis=0, keepdims=True)

    @pl.when(i == N_ROW_BLOCKS - 1)
    def _():
        se = se_ref[...]
        wavg = swa_ref[...] / se
        mean = sp_ref[...] / se
        var = spp_ref[...] / se - mean * mean
        vp = jnp.log(var + LOG_EPS)
        merged = jnp.concatenate([wavg, vp], axis=1)
        o_ref[...] = _dot(merged, hw_ref[...]) + hb_ref[...]


def _head(hs, p):
    cat = 4 * HID
    nvp = p["var_w"].shape[1]
    body = pl.pallas_call(
        _head_body,
        grid=(N_ROW_BLOCKS,),
        in_specs=[pl.BlockSpec((ROW_BLOCK, HID), lambda i: (i, 0))] * 4 + [
            pl.BlockSpec((cat, cat), lambda i: (0, 0)),
            pl.BlockSpec((1, cat), lambda i: (0, 0)),
            pl.BlockSpec((cat, cat), lambda i: (0, 0)),
            pl.BlockSpec((1, cat), lambda i: (0, 0)),
            pl.BlockSpec((cat, cat), lambda i: (0, 0)),
            pl.BlockSpec((1, cat), lambda i: (0, 0)),
            pl.BlockSpec((cat, 1), lambda i: (0, 0)),
            pl.BlockSpec((1, 1), lambda i: (0, 0)),
            pl.BlockSpec((cat, nvp), lambda i: (0, 0)),
            pl.BlockSpec((cat + nvp, 4), lambda i: (0, 0)),
            pl.BlockSpec((1, 4), lambda i: (0, 0)),
        ],
        out_specs=pl.BlockSpec((1, 4), lambda i: (0, 0)),
        out_shape=jax.ShapeDtypeStruct((1, 4), jnp.float32),
        scratch_shapes=[
            pltpu.VMEM((1, 1), jnp.float32),
            pltpu.VMEM((1, cat), jnp.float32),
            pltpu.VMEM((1, nvp), jnp.float32),
            pltpu.VMEM((1, nvp), jnp.float32),
        ],
        interpret=_INTERPRET,
    )
    return body(hs[0], hs[1], hs[2], hs[3],
                p["phi_w"], p["phi_b"].reshape(1, cat),
                p["attn_a_w"], p["attn_a_b"].reshape(1, cat),
                p["attn_b_w"], p["attn_b_b"].reshape(1, cat),
                p["attn_c_w"], p["attn_c_b"].reshape(1, 1),
                p["var_w"], p["head_w"], p["head_b"].reshape(1, 4))


def kernel(x, edge_index, params):
    p = params
    src = edge_index[0].astype(jnp.int32)
    dst = edge_index[1].astype(jnp.int32)
    h = _fc(x, p["fc_w"], p["fc_b"])
    hs = [h]
    for i, lp in enumerate(p["layers"]):
        tab = _msgtab(h, lp["t"])
        acc = _edge_pass(tab, src, dst)
        numer = jnp.concatenate([acc[0, :, :64], acc[1, :, :64]], axis=1)
        denom = jnp.concatenate([acc[0, :, 64:], acc[1, :, 64:]], axis=1)
        h = _post(h, numer, denom, lp, first=(i == 0))
        hs.append(h)
    return _head(hs, p)


# speedup vs baseline: 6.3916x; 1.0603x over previous
import functools

import jax
import jax.numpy as jnp
from jax import lax
from jax.experimental import pallas as pl
from jax.experimental.pallas import tpu as pltpu
from jax.experimental.pallas import tpu_sc as plsc

N_NODES = 10000
N_EDGES = 320000
HID = 128
GEN_EPS = 1e-7
LOG_EPS = 0.01
_PREC = lax.Precision.HIGHEST

ROW_BLOCK = 1000
N_ROW_BLOCKS = N_NODES // ROW_BLOCK

_INTERPRET = False


def _ln(y, g, b):
    mu = jnp.mean(y, axis=-1, keepdims=True)
    var = jnp.mean((y - mu) ** 2, axis=-1, keepdims=True)
    return (y - mu) / jnp.sqrt(var + 1e-5) * g + b


def _dot(a, b):
    return jnp.dot(a, b, preferred_element_type=jnp.float32, precision=_PREC)


def _fc_body(x_ref, w_ref, b_ref, o_ref):
    o_ref[...] = jax.nn.relu(_dot(x_ref[...], w_ref[...]) + b_ref[...])


def _fc(x, w, b):
    in_dim = x.shape[1]
    return pl.pallas_call(
        _fc_body,
        grid=(N_ROW_BLOCKS,),
        in_specs=[
            pl.BlockSpec((ROW_BLOCK, in_dim), lambda i: (i, 0)),
            pl.BlockSpec((in_dim, HID), lambda i: (0, 0)),
            pl.BlockSpec((1, HID), lambda i: (0, 0)),
        ],
        out_specs=pl.BlockSpec((ROW_BLOCK, HID), lambda i: (i, 0)),
        out_shape=jax.ShapeDtypeStruct((N_NODES, HID), jnp.float32),
        interpret=_INTERPRET,
    )(x, w, b.reshape(1, HID))


def _msgtab_body(t_ref, h_ref, o_ref):
    m = jax.nn.relu(h_ref[...]) + GEN_EPS
    e = jnp.exp(t_ref[0, 0] * m)
    em = e * m
    o_ref[...] = jnp.concatenate(
        [em[:, :64], e[:, :64], em[:, 64:], e[:, 64:]], axis=1)


def _msgtab(h, t):
    return pl.pallas_call(
        _msgtab_body,
        grid=(N_ROW_BLOCKS,),
        in_specs=[
            pl.BlockSpec((1, 1), lambda i: (0, 0)),
            pl.BlockSpec((ROW_BLOCK, HID), lambda i: (i, 0)),
        ],
        out_specs=pl.BlockSpec((ROW_BLOCK, 2 * HID), lambda i: (i, 0)),
        out_shape=jax.ShapeDtypeStruct((N_NODES, 2 * HID), jnp.float32),
        interpret=_INTERPRET,
    )(t.reshape(1, 1), h)


_SC_SUBCORES = 16
_EDGES_PER_SUB = N_EDGES // _SC_SUBCORES
_W = 80
_NWIN = _EDGES_PER_SUB // _W
_RS = 624
_RS_LAST = N_NODES - 15 * _RS


_EPS_PAD = 20480
_NWIN_P = _EPS_PAD // _W
_IG = 8
_NIG = _NWIN_P // _IG
_G = 4


def _sc_edge_body(tab_hbm, src_hbm, dst_hbm, zeros_hbm, out_hbm,
                  gixa, gixb, dsta, dstb, rows_v, acc_sh, isem, gsem, ssem):
    c = lax.axis_index("c")
    s = lax.axis_index("s")
    cv = jnp.broadcast_to(c, (16,)).astype(jnp.int32)
    r0 = s * _RS

    @pl.when(s < 15)
    def _():
        pltpu.sync_copy(zeros_hbm.at[pl.ds(r0, _RS)],
                        acc_sh.at[pl.ds(r0, _RS)])

    @pl.when(s == 15)
    def _():
        pltpu.sync_copy(zeros_hbm.at[pl.ds(15 * _RS, _RS_LAST)],
                        acc_sh.at[pl.ds(15 * _RS, _RS_LAST)])

    def idx_fire(ig, gix, dstw):
        pltpu.async_copy(src_hbm.at[s, pl.ds(ig * _IG, _IG)], gix, isem)
        pltpu.async_copy(dst_hbm.at[s, pl.ds(ig * _IG, _IG)], dstw, isem)

    def idx_wait(gix, dstw):
        pltpu.make_async_copy(src_hbm.at[s, pl.ds(0, _IG)], gix, isem).wait()
        pltpu.make_async_copy(dst_hbm.at[s, pl.ds(0, _IG)], dstw, isem).wait()
        for w in range(_IG):
            for j in range(0, _W, 16):
                gix[w, pl.ds(j, 16)] = gix[w, pl.ds(j, 16)] * 2 + cv

    def process(gix, dstw):
        for half in range(_IG // _G):
            gs = [pltpu.async_copy(tab_hbm.at[gix.at[half * _G + b]],
                                   rows_v.at[b], gsem) for b in range(_G)]
            for d in gs:
                d.wait()
            ss = [pltpu.async_copy(rows_v.at[b],
                                   acc_sh.at[dstw.at[half * _G + b]],
                                   ssem, add=True) for b in range(_G)]
            for d in ss:
                d.wait()

    idx_fire(0, gixa, dsta)
    plsc.subcore_barrier()

    @pl.loop(0, _NIG // 2)
    def _grp(u):
        idx_wait(gixa, dsta)
        idx_fire(2 * u + 1, gixb, dstb)
        process(gixa, dsta)
        idx_wait(gixb, dstb)

        @pl.when(u < _NIG // 2 - 1)
        def _():
            idx_fire(2 * u + 2, gixa, dsta)

        process(gixb, dstb)

    plsc.subcore_barrier()

    @pl.when(s < 15)
    def _():
        pltpu.sync_copy(acc_sh.at[pl.ds(r0, _RS)],
                        out_hbm.at[c, pl.ds(r0, _RS)])

    @pl.when(s == 15)
    def _():
        pltpu.sync_copy(acc_sh.at[pl.ds(15 * _RS, _RS_LAST)],
                        out_hbm.at[c, pl.ds(15 * _RS, _RS_LAST)])


def _edge_pass(tab, src_p, dst_p):
    t2 = jnp.concatenate(
        [tab.reshape(2 * N_NODES, HID), jnp.zeros((8, HID), jnp.float32)])
    zeros = jnp.zeros((N_NODES, HID), jnp.float32)
    mesh = plsc.VectorSubcoreMesh(core_axis_name="c", subcore_axis_name="s")
    f = pl.kernel(
        _sc_edge_body,
        out_type=jax.ShapeDtypeStruct((2, N_NODES, HID), jnp.float32),
        mesh=mesh,
        scratch_types=[
            pltpu.VMEM((_IG, _W), jnp.int32),
            pltpu.VMEM((_IG, _W), jnp.int32),
            pltpu.VMEM((_IG, _W), jnp.int32),
            pltpu.VMEM((_IG, _W), jnp.int32),
            pltpu.VMEM((_G, _W, HID), jnp.float32),
            pltpu.VMEM_SHARED((N_NODES, HID), jnp.float32),
            pltpu.SemaphoreType.DMA,
            pltpu.SemaphoreType.DMA,
            pltpu.SemaphoreType.DMA,
        ],
    )
    return f(t2, src_p, dst_p, zeros)


def _post_body(h_ref, num_ref, den_ref, w1_ref, b1_ref, g1_ref, bb1_ref,
               w2_ref, b2_ref, g2_ref, bb2_ref, o_ref, *, first):
    aggr = num_ref[...] / (den_ref[...] + 1e-16)
    out = h_ref[...] + aggr
    y = _dot(out, w1_ref[...]) + b1_ref[...]
    y = _ln(y, g1_ref[...], bb1_ref[...])
    y = jax.nn.relu(y)
    y = _dot(y, w2_ref[...]) + b2_ref[...]
    if first:
        o_ref[...] = y
    else:
        y = _ln(y, g2_ref[...], bb2_ref[...])
        o_ref[...] = h_ref[...] + jax.nn.relu(y)


def _post(h, numer, denom, lp, first):
    h2 = 2 * HID
    return pl.pallas_call(
        functools.partial(_post_body, first=first),
        grid=(N_ROW_BLOCKS,),
        in_specs=[
            pl.BlockSpec((ROW_BLOCK, HID), lambda i: (i, 0)),
            pl.BlockSpec((ROW_BLOCK, HID), lambda i: (i, 0)),
            pl.BlockSpec((ROW_BLOCK, HID), lambda i: (i, 0)),
            pl.BlockSpec((HID, h2), lambda i: (0, 0)),
            pl.BlockSpec((1, h2), lambda i: (0, 0)),
            pl.BlockSpec((1, h2), lambda i: (0, 0)),
            pl.BlockSpec((1, h2), lambda i: (0, 0)),
            pl.BlockSpec((h2, HID), lambda i: (0, 0)),
            pl.BlockSpec((1, HID), lambda i: (0, 0)),
            pl.BlockSpec((1, HID), lambda i: (0, 0)),
            pl.BlockSpec((1, HID), lambda i: (0, 0)),
        ],
        out_specs=pl.BlockSpec((ROW_BLOCK, HID), lambda i: (i, 0)),
        out_shape=jax.ShapeDtypeStruct((N_NODES, HID), jnp.float32),
        interpret=_INTERPRET,
    )(h, numer, denom,
      lp["w1"], lp["b1"].reshape(1, h2), lp["ln1_g"].reshape(1, h2),
      lp["ln1_b"].reshape(1, h2), lp["w2"], lp["b2"].reshape(1, HID),
      lp["ln_g"].reshape(1, HID), lp["ln_b"].reshape(1, HID))


def _head_body(h0_ref, h1_ref, h2_ref, h3_ref, phiw_ref, phib_ref,
               aw_ref, ab_ref, bw_ref, bb_ref, cw_ref, cb_ref, vw_ref,
               hw_ref, hb_ref, o_ref, se_ref, swa_ref, sp_ref, spp_ref):
    i = pl.program_id(0)

    @pl.when(i == 0)
    def _():
        se_ref[...] = jnp.zeros_like(se_ref)
        swa_ref[...] = jnp.zeros_like(swa_ref)
        sp_ref[...] = jnp.zeros_like(sp_ref)
        spp_ref[...] = jnp.zeros_like(spp_ref)

    xcat = jnp.concatenate(
        [h0_ref[...], h1_ref[...], h2_ref[...], h3_ref[...]], axis=1)
    hp = jax.nn.relu(_dot(xcat, phiw_ref[...]) + phib_ref[...])
    a = jnp.tanh(_dot(hp, aw_ref[...]) + ab_ref[...])
    g = jax.nn.sigmoid(_dot(hp, bw_ref[...]) + bb_ref[...])
    logit = _dot(a * g, cw_ref[...]) + cb_ref[...]
    e = jnp.exp(logit)
    proj = _dot(hp, vw_ref[...])
    se_ref[...] += jnp.sum(e).reshape(1, 1)
    swa_ref[...] += jnp.sum(e * hp, axis=0, keepdims=True)
    sp_ref[...] += jnp.sum(e * proj, axis=0, keepdims=True)
    spp_ref[...] += jnp.sum(e * proj * proj, axis=0, keepdims=True)

    @pl.when(i == N_ROW_BLOCKS - 1)
    def _():
        se = se_ref[...]
        wavg = swa_ref[...] / se
        mean = sp_ref[...] / se
        var = spp_ref[...] / se - mean * mean
        vp = jnp.log(var + LOG_EPS)
        merged = jnp.concatenate([wavg, vp], axis=1)
        o_ref[...] = _dot(merged, hw_ref[...]) + hb_ref[...]


def _head(hs, p):
    cat = 4 * HID
    nvp = p["var_w"].shape[1]
    body = pl.pallas_call(
        _head_body,
        grid=(N_ROW_BLOCKS,),
        in_specs=[pl.BlockSpec((ROW_BLOCK, HID), lambda i: (i, 0))] * 4 + [
            pl.BlockSpec((cat, cat), lambda i: (0, 0)),
            pl.BlockSpec((1, cat), lambda i: (0, 0)),
            pl.BlockSpec((cat, cat), lambda i: (0, 0)),
            pl.BlockSpec((1, cat), lambda i: (0, 0)),
            pl.BlockSpec((cat, cat), lambda i: (0, 0)),
            pl.BlockSpec((1, cat), lambda i: (0, 0)),
            pl.BlockSpec((cat, 1), lambda i: (0, 0)),
            pl.BlockSpec((1, 1), lambda i: (0, 0)),
            pl.BlockSpec((cat, nvp), lambda i: (0, 0)),
            pl.BlockSpec((cat + nvp, 4), lambda i: (0, 0)),
            pl.BlockSpec((1, 4), lambda i: (0, 0)),
        ],
        out_specs=pl.BlockSpec((1, 4), lambda i: (0, 0)),
        out_shape=jax.ShapeDtypeStruct((1, 4), jnp.float32),
        scratch_shapes=[
            pltpu.VMEM((1, 1), jnp.float32),
            pltpu.VMEM((1, cat), jnp.float32),
            pltpu.VMEM((1, nvp), jnp.float32),
            pltpu.VMEM((1, nvp), jnp.float32),
        ],
        interpret=_INTERPRET,
    )
    return body(hs[0], hs[1], hs[2], hs[3],
                p["phi_w"], p["phi_b"].reshape(1, cat),
                p["attn_a_w"], p["attn_a_b"].reshape(1, cat),
                p["attn_b_w"], p["attn_b_b"].reshape(1, cat),
                p["attn_c_w"], p["attn_c_b"].reshape(1, 1),
                p["var_w"], p["head_w"], p["head_b"].reshape(1, 4))


def kernel(x, edge_index, params):
    p = params
    pad = ((0, 0), (0, _EPS_PAD - _EDGES_PER_SUB))
    src_p = jnp.pad(
        edge_index[0].astype(jnp.int32).reshape(_SC_SUBCORES, _EDGES_PER_SUB),
        pad, constant_values=N_NODES).reshape(_SC_SUBCORES, _NWIN_P, _W)
    dst_p = jnp.pad(
        edge_index[1].astype(jnp.int32).reshape(_SC_SUBCORES, _EDGES_PER_SUB),
        pad, constant_values=0).reshape(_SC_SUBCORES, _NWIN_P, _W)
    h = _fc(x, p["fc_w"], p["fc_b"])
    hs = [h]
    for i, lp in enumerate(p["layers"]):
        tab = _msgtab(h, lp["t"])
        acc = _edge_pass(tab, src_p, dst_p)
        numer = jnp.concatenate([acc[0, :, :64], acc[1, :, :64]], axis=1)
        denom = jnp.concatenate([acc[0, :, 64:], acc[1, :, 64:]], axis=1)
        h = _post(h, numer, denom, lp, first=(i == 0))
        hs.append(h)
    return _head(hs, p)


# speedup vs baseline: 7.0714x; 1.1064x over previous
import functools

import jax
import jax.numpy as jnp
from jax import lax
from jax.experimental import pallas as pl
from jax.experimental.pallas import tpu as pltpu
from jax.experimental.pallas import tpu_sc as plsc

N_NODES = 10000
N_EDGES = 320000
HID = 128
GEN_EPS = 1e-7
LOG_EPS = 0.01
_PREC = None

ROW_BLOCK = 1000
N_ROW_BLOCKS = N_NODES // ROW_BLOCK

_INTERPRET = False


def _ln(y, g, b):
    mu = jnp.mean(y, axis=-1, keepdims=True)
    var = jnp.mean((y - mu) ** 2, axis=-1, keepdims=True)
    return (y - mu) / jnp.sqrt(var + 1e-5) * g + b


def _dot(a, b):
    return jnp.dot(a, b, preferred_element_type=jnp.float32, precision=_PREC)


def _fc_body(x_ref, w_ref, b_ref, o_ref):
    o_ref[...] = jax.nn.relu(_dot(x_ref[...], w_ref[...]) + b_ref[...])


def _fc(x, w, b):
    in_dim = x.shape[1]
    return pl.pallas_call(
        _fc_body,
        grid=(N_ROW_BLOCKS,),
        in_specs=[
            pl.BlockSpec((ROW_BLOCK, in_dim), lambda i: (i, 0)),
            pl.BlockSpec((in_dim, HID), lambda i: (0, 0)),
            pl.BlockSpec((1, HID), lambda i: (0, 0)),
        ],
        out_specs=pl.BlockSpec((ROW_BLOCK, HID), lambda i: (i, 0)),
        out_shape=jax.ShapeDtypeStruct((N_NODES, HID), jnp.float32),
        interpret=_INTERPRET,
    )(x, w, b.reshape(1, HID))


def _msgtab_body(t_ref, h_ref, o_ref):
    m = jax.nn.relu(h_ref[...]) + GEN_EPS
    e = jnp.exp(t_ref[0, 0] * m)
    em = e * m
    o_ref[...] = jnp.concatenate(
        [em[:, :64], e[:, :64], em[:, 64:], e[:, 64:]], axis=1)


def _msgtab(h, t):
    return pl.pallas_call(
        _msgtab_body,
        grid=(N_ROW_BLOCKS,),
        in_specs=[
            pl.BlockSpec((1, 1), lambda i: (0, 0)),
            pl.BlockSpec((ROW_BLOCK, HID), lambda i: (i, 0)),
        ],
        out_specs=pl.BlockSpec((ROW_BLOCK, 2 * HID), lambda i: (i, 0)),
        out_shape=jax.ShapeDtypeStruct((N_NODES, 2 * HID), jnp.float32),
        interpret=_INTERPRET,
    )(t.reshape(1, 1), h)


_SC_SUBCORES = 16
_EDGES_PER_SUB = N_EDGES // _SC_SUBCORES
_W = 80
_NWIN = _EDGES_PER_SUB // _W
_RS = 624
_RS_LAST = N_NODES - 15 * _RS


_EPS_PAD = 20480
_NWIN_P = _EPS_PAD // _W
_IG = 8
_NIG = _NWIN_P // _IG
_G = 4


def _sc_edge_body(tab_hbm, src_hbm, dst_hbm, zeros_hbm, out_hbm,
                  gixa, gixb, dsta, dstb, rows_v, acc_sh, isem, gsem, ssem):
    c = lax.axis_index("c")
    s = lax.axis_index("s")
    cv = jnp.broadcast_to(c, (16,)).astype(jnp.int32)
    r0 = s * _RS

    @pl.when(s < 15)
    def _():
        pltpu.sync_copy(zeros_hbm.at[pl.ds(r0, _RS)],
                        acc_sh.at[pl.ds(r0, _RS)])

    @pl.when(s == 15)
    def _():
        pltpu.sync_copy(zeros_hbm.at[pl.ds(15 * _RS, _RS_LAST)],
                        acc_sh.at[pl.ds(15 * _RS, _RS_LAST)])

    def idx_fire(ig, gix, dstw):
        pltpu.async_copy(src_hbm.at[s, pl.ds(ig * _IG, _IG)], gix, isem)
        pltpu.async_copy(dst_hbm.at[s, pl.ds(ig * _IG, _IG)], dstw, isem)

    def idx_wait(gix, dstw):
        pltpu.make_async_copy(src_hbm.at[s, pl.ds(0, _IG)], gix, isem).wait()
        pltpu.make_async_copy(dst_hbm.at[s, pl.ds(0, _IG)], dstw, isem).wait()
        for w in range(_IG):
            for j in range(0, _W, 16):
                gix[w, pl.ds(j, 16)] = gix[w, pl.ds(j, 16)] * 2 + cv

    def process(gix, dstw):
        for half in range(_IG // _G):
            gs = [pltpu.async_copy(tab_hbm.at[gix.at[half * _G + b]],
                                   rows_v.at[b], gsem) for b in range(_G)]
            for d in gs:
                d.wait()
            ss = [pltpu.async_copy(rows_v.at[b],
                                   acc_sh.at[dstw.at[half * _G + b]],
                                   ssem, add=True) for b in range(_G)]
            for d in ss:
                d.wait()

    idx_fire(0, gixa, dsta)
    plsc.subcore_barrier()

    @pl.loop(0, _NIG // 2)
    def _grp(u):
        idx_wait(gixa, dsta)
        idx_fire(2 * u + 1, gixb, dstb)
        process(gixa, dsta)
        idx_wait(gixb, dstb)

        @pl.when(u < _NIG // 2 - 1)
        def _():
            idx_fire(2 * u + 2, gixa, dsta)

        process(gixb, dstb)

    plsc.subcore_barrier()

    @pl.when(s < 15)
    def _():
        pltpu.sync_copy(acc_sh.at[pl.ds(r0, _RS)],
                        out_hbm.at[c, pl.ds(r0, _RS)])

    @pl.when(s == 15)
    def _():
        pltpu.sync_copy(acc_sh.at[pl.ds(15 * _RS, _RS_LAST)],
                        out_hbm.at[c, pl.ds(15 * _RS, _RS_LAST)])


def _edge_pass(tab, src_p, dst_p):
    t2 = jnp.concatenate(
        [tab.reshape(2 * N_NODES, HID), jnp.zeros((8, HID), jnp.float32)])
    zeros = jnp.zeros((N_NODES, HID), jnp.float32)
    mesh = plsc.VectorSubcoreMesh(core_axis_name="c", subcore_axis_name="s")
    f = pl.kernel(
        _sc_edge_body,
        out_type=jax.ShapeDtypeStruct((2, N_NODES, HID), jnp.float32),
        mesh=mesh,
        scratch_types=[
            pltpu.VMEM((_IG, _W), jnp.int32),
            pltpu.VMEM((_IG, _W), jnp.int32),
            pltpu.VMEM((_IG, _W), jnp.int32),
            pltpu.VMEM((_IG, _W), jnp.int32),
            pltpu.VMEM((_G, _W, HID), jnp.float32),
            pltpu.VMEM_SHARED((N_NODES, HID), jnp.float32),
            pltpu.SemaphoreType.DMA,
            pltpu.SemaphoreType.DMA,
            pltpu.SemaphoreType.DMA,
        ],
    )
    return f(t2, src_p, dst_p, zeros)


def _post_body(h_ref, num_ref, den_ref, w1_ref, b1_ref, g1_ref, bb1_ref,
               w2_ref, b2_ref, g2_ref, bb2_ref, o_ref, *, first):
    aggr = num_ref[...] / (den_ref[...] + 1e-16)
    out = h_ref[...] + aggr
    y = _dot(out, w1_ref[...]) + b1_ref[...]
    y = _ln(y, g1_ref[...], bb1_ref[...])
    y = jax.nn.relu(y)
    y = _dot(y, w2_ref[...]) + b2_ref[...]
    if first:
        o_ref[...] = y
    else:
        y = _ln(y, g2_ref[...], bb2_ref[...])
        o_ref[...] = h_ref[...] + jax.nn.relu(y)


def _post(h, numer, denom, lp, first):
    h2 = 2 * HID
    return pl.pallas_call(
        functools.partial(_post_body, first=first),
        grid=(N_ROW_BLOCKS,),
        in_specs=[
            pl.BlockSpec((ROW_BLOCK, HID), lambda i: (i, 0)),
            pl.BlockSpec((ROW_BLOCK, HID), lambda i: (i, 0)),
            pl.BlockSpec((ROW_BLOCK, HID), lambda i: (i, 0)),
            pl.BlockSpec((HID, h2), lambda i: (0, 0)),
            pl.BlockSpec((1, h2), lambda i: (0, 0)),
            pl.BlockSpec((1, h2), lambda i: (0, 0)),
            pl.BlockSpec((1, h2), lambda i: (0, 0)),
            pl.BlockSpec((h2, HID), lambda i: (0, 0)),
            pl.BlockSpec((1, HID), lambda i: (0, 0)),
            pl.BlockSpec((1, HID), lambda i: (0, 0)),
            pl.BlockSpec((1, HID), lambda i: (0, 0)),
        ],
        out_specs=pl.BlockSpec((ROW_BLOCK, HID), lambda i: (i, 0)),
        out_shape=jax.ShapeDtypeStruct((N_NODES, HID), jnp.float32),
        interpret=_INTERPRET,
    )(h, numer, denom,
      lp["w1"], lp["b1"].reshape(1, h2), lp["ln1_g"].reshape(1, h2),
      lp["ln1_b"].reshape(1, h2), lp["w2"], lp["b2"].reshape(1, HID),
      lp["ln_g"].reshape(1, HID), lp["ln_b"].reshape(1, HID))


def _head_body(h0_ref, h1_ref, h2_ref, h3_ref, phiw_ref, phib_ref,
               aw_ref, ab_ref, bw_ref, bb_ref, cw_ref, cb_ref, vw_ref,
               hw_ref, hb_ref, o_ref, se_ref, swa_ref, sp_ref, spp_ref):
    i = pl.program_id(0)

    @pl.when(i == 0)
    def _():
        se_ref[...] = jnp.zeros_like(se_ref)
        swa_ref[...] = jnp.zeros_like(swa_ref)
        sp_ref[...] = jnp.zeros_like(sp_ref)
        spp_ref[...] = jnp.zeros_like(spp_ref)

    xcat = jnp.concatenate(
        [h0_ref[...], h1_ref[...], h2_ref[...], h3_ref[...]], axis=1)
    hp = jax.nn.relu(_dot(xcat, phiw_ref[...]) + phib_ref[...])
    a = jnp.tanh(_dot(hp, aw_ref[...]) + ab_ref[...])
    g = jax.nn.sigmoid(_dot(hp, bw_ref[...]) + bb_ref[...])
    logit = _dot(a * g, cw_ref[...]) + cb_ref[...]
    e = jnp.exp(logit)
    proj = _dot(hp, vw_ref[...])
    se_ref[...] += jnp.sum(e).reshape(1, 1)
    swa_ref[...] += jnp.sum(e * hp, axis=0, keepdims=True)
    sp_ref[...] += jnp.sum(e * proj, axis=0, keepdims=True)
    spp_ref[...] += jnp.sum(e * proj * proj, axis=0, keepdims=True)

    @pl.when(i == N_ROW_BLOCKS - 1)
    def _():
        se = se_ref[...]
        wavg = swa_ref[...] / se
        mean = sp_ref[...] / se
        var = spp_ref[...] / se - mean * mean
        vp = jnp.log(var + LOG_EPS)
        merged = jnp.concatenate([wavg, vp], axis=1)
        o_ref[...] = _dot(merged, hw_ref[...]) + hb_ref[...]


def _head(hs, p):
    cat = 4 * HID
    nvp = p["var_w"].shape[1]
    body = pl.pallas_call(
        _head_body,
        grid=(N_ROW_BLOCKS,),
        in_specs=[pl.BlockSpec((ROW_BLOCK, HID), lambda i: (i, 0))] * 4 + [
            pl.BlockSpec((cat, cat), lambda i: (0, 0)),
            pl.BlockSpec((1, cat), lambda i: (0, 0)),
            pl.BlockSpec((cat, cat), lambda i: (0, 0)),
            pl.BlockSpec((1, cat), lambda i: (0, 0)),
            pl.BlockSpec((cat, cat), lambda i: (0, 0)),
            pl.BlockSpec((1, cat), lambda i: (0, 0)),
            pl.BlockSpec((cat, 1), lambda i: (0, 0)),
            pl.BlockSpec((1, 1), lambda i: (0, 0)),
            pl.BlockSpec((cat, nvp), lambda i: (0, 0)),
            pl.BlockSpec((cat + nvp, 4), lambda i: (0, 0)),
            pl.BlockSpec((1, 4), lambda i: (0, 0)),
        ],
        out_specs=pl.BlockSpec((1, 4), lambda i: (0, 0)),
        out_shape=jax.ShapeDtypeStruct((1, 4), jnp.float32),
        scratch_shapes=[
            pltpu.VMEM((1, 1), jnp.float32),
            pltpu.VMEM((1, cat), jnp.float32),
            pltpu.VMEM((1, nvp), jnp.float32),
            pltpu.VMEM((1, nvp), jnp.float32),
        ],
        interpret=_INTERPRET,
    )
    return body(hs[0], hs[1], hs[2], hs[3],
                p["phi_w"], p["phi_b"].reshape(1, cat),
                p["attn_a_w"], p["attn_a_b"].reshape(1, cat),
                p["attn_b_w"], p["attn_b_b"].reshape(1, cat),
                p["attn_c_w"], p["attn_c_b"].reshape(1, 1),
                p["var_w"], p["head_w"], p["head_b"].reshape(1, 4))


def kernel(x, edge_index, params):
    p = params
    pad = ((0, 0), (0, _EPS_PAD - _EDGES_PER_SUB))
    src_p = jnp.pad(
        edge_index[0].astype(jnp.int32).reshape(_SC_SUBCORES, _EDGES_PER_SUB),
        pad, constant_values=N_NODES).reshape(_SC_SUBCORES, _NWIN_P, _W)
    dst_p = jnp.pad(
        edge_index[1].astype(jnp.int32).reshape(_SC_SUBCORES, _EDGES_PER_SUB),
        pad, constant_values=0).reshape(_SC_SUBCORES, _NWIN_P, _W)
    h = _fc(x, p["fc_w"], p["fc_b"])
    hs = [h]
    for i, lp in enumerate(p["layers"]):
        tab = _msgtab(h, lp["t"])
        acc = _edge_pass(tab, src_p, dst_p)
        numer = jnp.concatenate([acc[0, :, :64], acc[1, :, :64]], axis=1)
        denom = jnp.concatenate([acc[0, :, 64:], acc[1, :, 64:]], axis=1)
        h = _post(h, numer, denom, lp, first=(i == 0))
        hs.append(h)
    return _head(hs, p)


# speedup vs baseline: 7.6152x; 1.0769x over previous
import functools

import jax
import jax.numpy as jnp
from jax import lax
from jax.experimental import pallas as pl
from jax.experimental.pallas import tpu as pltpu
from jax.experimental.pallas import tpu_sc as plsc

N_NODES = 10000
N_EDGES = 320000
HID = 128
GEN_EPS = 1e-7
LOG_EPS = 0.01
_PREC = None

ROW_BLOCK = 1000
N_ROW_BLOCKS = N_NODES // ROW_BLOCK

_INTERPRET = False


def _ln(y, g, b):
    mu = jnp.mean(y, axis=-1, keepdims=True)
    var = jnp.mean((y - mu) ** 2, axis=-1, keepdims=True)
    return (y - mu) / jnp.sqrt(var + 1e-5) * g + b


def _dot(a, b):
    return jnp.dot(a, b, preferred_element_type=jnp.float32, precision=_PREC)


def _fc_body(x_ref, w_ref, b_ref, o_ref):
    o_ref[...] = jax.nn.relu(_dot(x_ref[...], w_ref[...]) + b_ref[...])


def _fc(x, w, b):
    in_dim = x.shape[1]
    return pl.pallas_call(
        _fc_body,
        grid=(N_ROW_BLOCKS,),
        in_specs=[
            pl.BlockSpec((ROW_BLOCK, in_dim), lambda i: (i, 0)),
            pl.BlockSpec((in_dim, HID), lambda i: (0, 0)),
            pl.BlockSpec((1, HID), lambda i: (0, 0)),
        ],
        out_specs=pl.BlockSpec((ROW_BLOCK, HID), lambda i: (i, 0)),
        out_shape=jax.ShapeDtypeStruct((N_NODES, HID), jnp.float32),
        interpret=_INTERPRET,
    )(x, w, b.reshape(1, HID))


def _msgtab_body(t_ref, h_ref, o_ref):
    m = jax.nn.relu(h_ref[...]) + GEN_EPS
    e = jnp.exp(t_ref[0, 0] * m)
    em = e * m
    o_ref[...] = jnp.concatenate(
        [em[:, :64], e[:, :64], em[:, 64:], e[:, 64:]], axis=1)


def _msgtab(h, t):
    return pl.pallas_call(
        _msgtab_body,
        grid=(N_ROW_BLOCKS,),
        in_specs=[
            pl.BlockSpec((1, 1), lambda i: (0, 0)),
            pl.BlockSpec((ROW_BLOCK, HID), lambda i: (i, 0)),
        ],
        out_specs=pl.BlockSpec((ROW_BLOCK, 2 * HID), lambda i: (i, 0)),
        out_shape=jax.ShapeDtypeStruct((N_NODES, 2 * HID), jnp.float32),
        interpret=_INTERPRET,
    )(t.reshape(1, 1), h)


_SC_SUBCORES = 16
_EDGES_PER_SUB = N_EDGES // _SC_SUBCORES
_W = 80
_NWIN = _EDGES_PER_SUB // _W
_RS = 624
_RS_LAST = N_NODES - 15 * _RS


_EPS_PAD = 20480
_NWIN_P = _EPS_PAD // _W
_IG = 8
_NIG = _NWIN_P // _IG
_G = 4


def _sc_edge_body(tab_hbm, src_hbm, dst_hbm, zeros_hbm, out_hbm,
                  gixa, gixb, dsta, dstb, rows_v, acc_sh, isem, gsem, ssem):
    c = lax.axis_index("c")
    s = lax.axis_index("s")
    cv = jnp.broadcast_to(c, (16,)).astype(jnp.int32)
    r0 = s * _RS

    @pl.when(s < 15)
    def _():
        pltpu.sync_copy(zeros_hbm.at[pl.ds(r0, _RS)],
                        acc_sh.at[pl.ds(r0, _RS)])

    @pl.when(s == 15)
    def _():
        pltpu.sync_copy(zeros_hbm.at[pl.ds(15 * _RS, _RS_LAST)],
                        acc_sh.at[pl.ds(15 * _RS, _RS_LAST)])

    def idx_fire(ig, gix, dstw):
        pltpu.async_copy(src_hbm.at[s, pl.ds(ig * _IG, _IG)], gix, isem)
        pltpu.async_copy(dst_hbm.at[s, pl.ds(ig * _IG, _IG)], dstw, isem)

    def idx_wait(gix, dstw):
        pltpu.make_async_copy(src_hbm.at[s, pl.ds(0, _IG)], gix, isem).wait()
        pltpu.make_async_copy(dst_hbm.at[s, pl.ds(0, _IG)], dstw, isem).wait()
        for w in range(_IG):
            for j in range(0, _W, 16):
                gix[w, pl.ds(j, 16)] = gix[w, pl.ds(j, 16)] * 2 + cv

    def process(gix, dstw):
        def g(w, b):
            return pltpu.async_copy(tab_hbm.at[gix.at[w]], rows_v.at[b],
                                    gsem)

        def sc(w, b):
            return pltpu.async_copy(rows_v.at[b], acc_sh.at[dstw.at[w]],
                                    ssem, add=True)

        g0, g1, g2, g3 = g(0, 0), g(1, 1), g(2, 2), g(3, 3)
        g0.wait(); g1.wait()
        s0, s1 = sc(0, 0), sc(1, 1)
        g2.wait(); g3.wait()
        s2, s3 = sc(2, 2), sc(3, 3)
        s0.wait(); s1.wait()
        g4, g5 = g(4, 0), g(5, 1)
        s2.wait(); s3.wait()
        g6, g7 = g(6, 2), g(7, 3)
        g4.wait(); g5.wait()
        s4, s5 = sc(4, 0), sc(5, 1)
        g6.wait(); g7.wait()
        s6, s7 = sc(6, 2), sc(7, 3)
        s4.wait(); s5.wait(); s6.wait(); s7.wait()

    idx_fire(0, gixa, dsta)
    plsc.subcore_barrier()

    @pl.loop(0, _NIG // 2)
    def _grp(u):
        idx_wait(gixa, dsta)
        idx_fire(2 * u + 1, gixb, dstb)
        process(gixa, dsta)
        idx_wait(gixb, dstb)

        @pl.when(u < _NIG // 2 - 1)
        def _():
            idx_fire(2 * u + 2, gixa, dsta)

        process(gixb, dstb)

    plsc.subcore_barrier()

    @pl.when(s < 15)
    def _():
        pltpu.sync_copy(acc_sh.at[pl.ds(r0, _RS)],
                        out_hbm.at[c, pl.ds(r0, _RS)])

    @pl.when(s == 15)
    def _():
        pltpu.sync_copy(acc_sh.at[pl.ds(15 * _RS, _RS_LAST)],
                        out_hbm.at[c, pl.ds(15 * _RS, _RS_LAST)])


def _edge_pass(tab, src_p, dst_p):
    t2 = jnp.concatenate(
        [tab.reshape(2 * N_NODES, HID), jnp.zeros((8, HID), jnp.float32)])
    zeros = jnp.zeros((N_NODES, HID), jnp.float32)
    mesh = plsc.VectorSubcoreMesh(core_axis_name="c", subcore_axis_name="s")
    f = pl.kernel(
        _sc_edge_body,
        out_type=jax.ShapeDtypeStruct((2, N_NODES, HID), jnp.float32),
        mesh=mesh,
        scratch_types=[
            pltpu.VMEM((_IG, _W), jnp.int32),
            pltpu.VMEM((_IG, _W), jnp.int32),
            pltpu.VMEM((_IG, _W), jnp.int32),
            pltpu.VMEM((_IG, _W), jnp.int32),
            pltpu.VMEM((_G, _W, HID), jnp.float32),
            pltpu.VMEM_SHARED((N_NODES, HID), jnp.float32),
            pltpu.SemaphoreType.DMA,
            pltpu.SemaphoreType.DMA,
            pltpu.SemaphoreType.DMA,
        ],
    )
    return f(t2, src_p, dst_p, zeros)


def _post_body(h_ref, num_ref, den_ref, w1_ref, b1_ref, g1_ref, bb1_ref,
               w2_ref, b2_ref, g2_ref, bb2_ref, o_ref, *, first):
    aggr = num_ref[...] / (den_ref[...] + 1e-16)
    out = h_ref[...] + aggr
    y = _dot(out, w1_ref[...]) + b1_ref[...]
    y = _ln(y, g1_ref[...], bb1_ref[...])
    y = jax.nn.relu(y)
    y = _dot(y, w2_ref[...]) + b2_ref[...]
    if first:
        o_ref[...] = y
    else:
        y = _ln(y, g2_ref[...], bb2_ref[...])
        o_ref[...] = h_ref[...] + jax.nn.relu(y)


def _post(h, numer, denom, lp, first):
    h2 = 2 * HID
    return pl.pallas_call(
        functools.partial(_post_body, first=first),
        grid=(N_ROW_BLOCKS,),
        in_specs=[
            pl.BlockSpec((ROW_BLOCK, HID), lambda i: (i, 0)),
            pl.BlockSpec((ROW_BLOCK, HID), lambda i: (i, 0)),
            pl.BlockSpec((ROW_BLOCK, HID), lambda i: (i, 0)),
            pl.BlockSpec((HID, h2), lambda i: (0, 0)),
            pl.BlockSpec((1, h2), lambda i: (0, 0)),
            pl.BlockSpec((1, h2), lambda i: (0, 0)),
            pl.BlockSpec((1, h2), lambda i: (0, 0)),
            pl.BlockSpec((h2, HID), lambda i: (0, 0)),
            pl.BlockSpec((1, HID), lambda i: (0, 0)),
            pl.BlockSpec((1, HID), lambda i: (0, 0)),
            pl.BlockSpec((1, HID), lambda i: (0, 0)),
        ],
        out_specs=pl.BlockSpec((ROW_BLOCK, HID), lambda i: (i, 0)),
        out_shape=jax.ShapeDtypeStruct((N_NODES, HID), jnp.float32),
        interpret=_INTERPRET,
    )(h, numer, denom,
      lp["w1"], lp["b1"].reshape(1, h2), lp["ln1_g"].reshape(1, h2),
      lp["ln1_b"].reshape(1, h2), lp["w2"], lp["b2"].reshape(1, HID),
      lp["ln_g"].reshape(1, HID), lp["ln_b"].reshape(1, HID))


def _head_body(h0_ref, h1_ref, h2_ref, h3_ref, phiw_ref, phib_ref,
               aw_ref, ab_ref, bw_ref, bb_ref, cw_ref, cb_ref, vw_ref,
               hw_ref, hb_ref, o_ref, se_ref, swa_ref, sp_ref, spp_ref):
    i = pl.program_id(0)

    @pl.when(i == 0)
    def _():
        se_ref[...] = jnp.zeros_like(se_ref)
        swa_ref[...] = jnp.zeros_like(swa_ref)
        sp_ref[...] = jnp.zeros_like(sp_ref)
        spp_ref[...] = jnp.zeros_like(spp_ref)

    xcat = jnp.concatenate(
        [h0_ref[...], h1_ref[...], h2_ref[...], h3_ref[...]], axis=1)
    hp = jax.nn.relu(_dot(xcat, phiw_ref[...]) + phib_ref[...])
    a = jnp.tanh(_dot(hp, aw_ref[...]) + ab_ref[...])
    g = jax.nn.sigmoid(_dot(hp, bw_ref[...]) + bb_ref[...])
    logit = _dot(a * g, cw_ref[...]) + cb_ref[...]
    e = jnp.exp(logit)
    proj = _dot(hp, vw_ref[...])
    se_ref[...] += jnp.sum(e).reshape(1, 1)
    swa_ref[...] += jnp.sum(e * hp, axis=0, keepdims=True)
    sp_ref[...] += jnp.sum(e * proj, axis=0, keepdims=True)
    spp_ref[...] += jnp.sum(e * proj * proj, axis=0, keepdims=True)

    @pl.when(i == N_ROW_BLOCKS - 1)
    def _():
        se = se_ref[...]
        wavg = swa_ref[...] / se
        mean = sp_ref[...] / se
        var = spp_ref[...] / se - mean * mean
        vp = jnp.log(var + LOG_EPS)
        merged = jnp.concatenate([wavg, vp], axis=1)
        o_ref[...] = _dot(merged, hw_ref[...]) + hb_ref[...]


def _head(hs, p):
    cat = 4 * HID
    nvp = p["var_w"].shape[1]
    body = pl.pallas_call(
        _head_body,
        grid=(N_ROW_BLOCKS,),
        in_specs=[pl.BlockSpec((ROW_BLOCK, HID), lambda i: (i, 0))] * 4 + [
            pl.BlockSpec((cat, cat), lambda i: (0, 0)),
            pl.BlockSpec((1, cat), lambda i: (0, 0)),
            pl.BlockSpec((cat, cat), lambda i: (0, 0)),
            pl.BlockSpec((1, cat), lambda i: (0, 0)),
            pl.BlockSpec((cat, cat), lambda i: (0, 0)),
            pl.BlockSpec((1, cat), lambda i: (0, 0)),
            pl.BlockSpec((cat, 1), lambda i: (0, 0)),
            pl.BlockSpec((1, 1), lambda i: (0, 0)),
            pl.BlockSpec((cat, nvp), lambda i: (0, 0)),
            pl.BlockSpec((cat + nvp, 4), lambda i: (0, 0)),
            pl.BlockSpec((1, 4), lambda i: (0, 0)),
        ],
        out_specs=pl.BlockSpec((1, 4), lambda i: (0, 0)),
        out_shape=jax.ShapeDtypeStruct((1, 4), jnp.float32),
        scratch_shapes=[
            pltpu.VMEM((1, 1), jnp.float32),
            pltpu.VMEM((1, cat), jnp.float32),
            pltpu.VMEM((1, nvp), jnp.float32),
            pltpu.VMEM((1, nvp), jnp.float32),
        ],
        interpret=_INTERPRET,
    )
    return body(hs[0], hs[1], hs[2], hs[3],
                p["phi_w"], p["phi_b"].reshape(1, cat),
                p["attn_a_w"], p["attn_a_b"].reshape(1, cat),
                p["attn_b_w"], p["attn_b_b"].reshape(1, cat),
                p["attn_c_w"], p["attn_c_b"].reshape(1, 1),
                p["var_w"], p["head_w"], p["head_b"].reshape(1, 4))


def kernel(x, edge_index, params):
    p = params
    pad = ((0, 0), (0, _EPS_PAD - _EDGES_PER_SUB))
    src_p = jnp.pad(
        edge_index[0].astype(jnp.int32).reshape(_SC_SUBCORES, _EDGES_PER_SUB),
        pad, constant_values=N_NODES).reshape(_SC_SUBCORES, _NWIN_P, _W)
    dst_p = jnp.pad(
        edge_index[1].astype(jnp.int32).reshape(_SC_SUBCORES, _EDGES_PER_SUB),
        pad, constant_values=0).reshape(_SC_SUBCORES, _NWIN_P, _W)
    h = _fc(x, p["fc_w"], p["fc_b"])
    hs = [h]
    for i, lp in enumerate(p["layers"]):
        tab = _msgtab(h, lp["t"])
        acc = _edge_pass(tab, src_p, dst_p)
        numer = jnp.concatenate([acc[0, :, :64], acc[1, :, :64]], axis=1)
        denom = jnp.concatenate([acc[0, :, 64:], acc[1, :, 64:]], axis=1)
        h = _post(h, numer, denom, lp, first=(i == 0))
        hs.append(h)
    return _head(hs, p)


# speedup vs baseline: 7.6550x; 1.0052x over previous
import functools

import jax
import jax.numpy as jnp
from jax import lax
from jax.experimental import pallas as pl
from jax.experimental.pallas import tpu as pltpu
from jax.experimental.pallas import tpu_sc as plsc

N_NODES = 10000
N_EDGES = 320000
HID = 128
GEN_EPS = 1e-7
LOG_EPS = 0.01
_PREC = None

ROW_BLOCK = 1000
N_ROW_BLOCKS = N_NODES // ROW_BLOCK

_INTERPRET = False


def _ln(y, g, b):
    mu = jnp.mean(y, axis=-1, keepdims=True)
    var = jnp.mean((y - mu) ** 2, axis=-1, keepdims=True)
    return (y - mu) / jnp.sqrt(var + 1e-5) * g + b


def _dot(a, b):
    return jnp.dot(a, b, preferred_element_type=jnp.float32, precision=_PREC)


def _tab_from_h(h, t):
    m = h + GEN_EPS
    e = jnp.exp(t * m)
    em = e * m
    return jnp.concatenate(
        [em[:, :64], e[:, :64], em[:, 64:], e[:, 64:]], axis=1)


def _fc_body(t_ref, x_ref, w_ref, b_ref, o_ref, tab_ref):
    h = jax.nn.relu(_dot(x_ref[...], w_ref[...]) + b_ref[...])
    o_ref[...] = h
    tab_ref[...] = _tab_from_h(h, t_ref[0, 0])


def _fc(x, w, b, t):
    in_dim = x.shape[1]
    return pl.pallas_call(
        _fc_body,
        grid=(N_ROW_BLOCKS,),
        in_specs=[
            pl.BlockSpec((1, 1), lambda i: (0, 0)),
            pl.BlockSpec((ROW_BLOCK, in_dim), lambda i: (i, 0)),
            pl.BlockSpec((in_dim, HID), lambda i: (0, 0)),
            pl.BlockSpec((1, HID), lambda i: (0, 0)),
        ],
        out_specs=[
            pl.BlockSpec((ROW_BLOCK, HID), lambda i: (i, 0)),
            pl.BlockSpec((ROW_BLOCK, 2 * HID), lambda i: (i, 0)),
        ],
        out_shape=[
            jax.ShapeDtypeStruct((N_NODES, HID), jnp.float32),
            jax.ShapeDtypeStruct((N_NODES, 2 * HID), jnp.float32),
        ],
        interpret=_INTERPRET,
    )(t.reshape(1, 1), x, w, b.reshape(1, HID))


_SC_SUBCORES = 16
_EDGES_PER_SUB = N_EDGES // _SC_SUBCORES
_W = 80
_NWIN = _EDGES_PER_SUB // _W
_RS = 624
_RS_LAST = N_NODES - 15 * _RS


_EPS_PAD = 20480
_NWIN_P = _EPS_PAD // _W
_IG = 8
_NIG = _NWIN_P // _IG
_G = 4


def _sc_edge_body(tab_hbm, src_hbm, dst_hbm, zeros_hbm, out_hbm,
                  gixa, gixb, dsta, dstb, rows_v, acc_sh, isem, gsem, ssem):
    c = lax.axis_index("c")
    s = lax.axis_index("s")
    cv = jnp.broadcast_to(c, (16,)).astype(jnp.int32)
    r0 = s * _RS

    @pl.when(s < 15)
    def _():
        pltpu.sync_copy(zeros_hbm.at[pl.ds(r0, _RS)],
                        acc_sh.at[pl.ds(r0, _RS)])

    @pl.when(s == 15)
    def _():
        pltpu.sync_copy(zeros_hbm.at[pl.ds(15 * _RS, _RS_LAST)],
                        acc_sh.at[pl.ds(15 * _RS, _RS_LAST)])

    def idx_fire(ig, gix, dstw):
        pltpu.async_copy(src_hbm.at[s, pl.ds(ig * _IG, _IG)], gix, isem)
        pltpu.async_copy(dst_hbm.at[s, pl.ds(ig * _IG, _IG)], dstw, isem)

    def idx_wait(gix, dstw):
        pltpu.make_async_copy(src_hbm.at[s, pl.ds(0, _IG)], gix, isem).wait()
        pltpu.make_async_copy(dst_hbm.at[s, pl.ds(0, _IG)], dstw, isem).wait()
        for w in range(_IG):
            for j in range(0, _W, 16):
                gix[w, pl.ds(j, 16)] = gix[w, pl.ds(j, 16)] * 2 + cv

    def process(gix, dstw):
        def g(w, b):
            return pltpu.async_copy(tab_hbm.at[gix.at[w]], rows_v.at[b],
                                    gsem)

        def sc(w, b):
            return pltpu.async_copy(rows_v.at[b], acc_sh.at[dstw.at[w]],
                                    ssem, add=True)

        g0, g1, g2, g3 = g(0, 0), g(1, 1), g(2, 2), g(3, 3)
        g0.wait(); g1.wait()
        s0, s1 = sc(0, 0), sc(1, 1)
        g2.wait(); g3.wait()
        s2, s3 = sc(2, 2), sc(3, 3)
        s0.wait(); s1.wait()
        g4, g5 = g(4, 0), g(5, 1)
        s2.wait(); s3.wait()
        g6, g7 = g(6, 2), g(7, 3)
        g4.wait(); g5.wait()
        s4, s5 = sc(4, 0), sc(5, 1)
        g6.wait(); g7.wait()
        s6, s7 = sc(6, 2), sc(7, 3)
        s4.wait(); s5.wait(); s6.wait(); s7.wait()

    idx_fire(0, gixa, dsta)
    plsc.subcore_barrier()

    @pl.loop(0, _NIG // 2)
    def _grp(u):
        idx_wait(gixa, dsta)
        idx_fire(2 * u + 1, gixb, dstb)
        process(gixa, dsta)
        idx_wait(gixb, dstb)

        @pl.when(u < _NIG // 2 - 1)
        def _():
            idx_fire(2 * u + 2, gixa, dsta)

        process(gixb, dstb)

    plsc.subcore_barrier()

    @pl.when(s < 15)
    def _():
        pltpu.sync_copy(acc_sh.at[pl.ds(r0, _RS)],
                        out_hbm.at[c, pl.ds(r0, _RS)])

    @pl.when(s == 15)
    def _():
        pltpu.sync_copy(acc_sh.at[pl.ds(15 * _RS, _RS_LAST)],
                        out_hbm.at[c, pl.ds(15 * _RS, _RS_LAST)])


def _edge_pass(tab, src_p, dst_p):
    t2 = jnp.concatenate(
        [tab.reshape(2 * N_NODES, HID), jnp.zeros((8, HID), jnp.float32)])
    zeros = jnp.zeros((N_NODES, HID), jnp.float32)
    mesh = plsc.VectorSubcoreMesh(core_axis_name="c", subcore_axis_name="s")
    f = pl.kernel(
        _sc_edge_body,
        out_type=jax.ShapeDtypeStruct((2, N_NODES, HID), jnp.float32),
        mesh=mesh,
        scratch_types=[
            pltpu.VMEM((_IG, _W), jnp.int32),
            pltpu.VMEM((_IG, _W), jnp.int32),
            pltpu.VMEM((_IG, _W), jnp.int32),
            pltpu.VMEM((_IG, _W), jnp.int32),
            pltpu.VMEM((_G, _W, HID), jnp.float32),
            pltpu.VMEM_SHARED((N_NODES, HID), jnp.float32),
            pltpu.SemaphoreType.DMA,
            pltpu.SemaphoreType.DMA,
            pltpu.SemaphoreType.DMA,
        ],
    )
    return f(t2, src_p, dst_p, zeros)


def _post_body(t_ref, h_ref, num_ref, den_ref, w1_ref, b1_ref, g1_ref,
               bb1_ref, w2_ref, b2_ref, g2_ref, bb2_ref, o_ref, *maybe_tab,
               first, with_tab):
    aggr = num_ref[...] / (den_ref[...] + 1e-16)
    out = h_ref[...] + aggr
    y = _dot(out, w1_ref[...]) + b1_ref[...]
    y = _ln(y, g1_ref[...], bb1_ref[...])
    y = jax.nn.relu(y)
    y = _dot(y, w2_ref[...]) + b2_ref[...]
    if first:
        h_new = y
    else:
        y = _ln(y, g2_ref[...], bb2_ref[...])
        h_new = h_ref[...] + jax.nn.relu(y)
    o_ref[...] = h_new
    if with_tab:
        maybe_tab[0][...] = _tab_from_h(jax.nn.relu(h_new), t_ref[0, 0])


def _post(h, numer, denom, lp, first, t_next):
    h2 = 2 * HID
    with_tab = t_next is not None
    out_specs = [pl.BlockSpec((ROW_BLOCK, HID), lambda i: (i, 0))]
    out_shape = [jax.ShapeDtypeStruct((N_NODES, HID), jnp.float32)]
    if with_tab:
        out_specs.append(pl.BlockSpec((ROW_BLOCK, 2 * HID), lambda i: (i, 0)))
        out_shape.append(
            jax.ShapeDtypeStruct((N_NODES, 2 * HID), jnp.float32))
    tv = t_next if with_tab else lp["t"]
    res = pl.pallas_call(
        functools.partial(_post_body, first=first, with_tab=with_tab),
        grid=(N_ROW_BLOCKS,),
        in_specs=[
            pl.BlockSpec((1, 1), lambda i: (0, 0)),
            pl.BlockSpec((ROW_BLOCK, HID), lambda i: (i, 0)),
            pl.BlockSpec((ROW_BLOCK, HID), lambda i: (i, 0)),
            pl.BlockSpec((ROW_BLOCK, HID), lambda i: (i, 0)),
            pl.BlockSpec((HID, h2), lambda i: (0, 0)),
            pl.BlockSpec((1, h2), lambda i: (0, 0)),
            pl.BlockSpec((1, h2), lambda i: (0, 0)),
            pl.BlockSpec((1, h2), lambda i: (0, 0)),
            pl.BlockSpec((h2, HID), lambda i: (0, 0)),
            pl.BlockSpec((1, HID), lambda i: (0, 0)),
            pl.BlockSpec((1, HID), lambda i: (0, 0)),
            pl.BlockSpec((1, HID), lambda i: (0, 0)),
        ],
        out_specs=out_specs,
        out_shape=out_shape,
        interpret=_INTERPRET,
    )(tv.reshape(1, 1), h, numer, denom,
      lp["w1"], lp["b1"].reshape(1, h2), lp["ln1_g"].reshape(1, h2),
      lp["ln1_b"].reshape(1, h2), lp["w2"], lp["b2"].reshape(1, HID),
      lp["ln_g"].reshape(1, HID), lp["ln_b"].reshape(1, HID))
    return res if with_tab else (res[0], None)


def _head_body(h0_ref, h1_ref, h2_ref, h3_ref, phiw_ref, phib_ref,
               aw_ref, ab_ref, bw_ref, bb_ref, cw_ref, cb_ref, vw_ref,
               hw_ref, hb_ref, o_ref, se_ref, swa_ref, sp_ref, spp_ref):
    i = pl.program_id(0)

    @pl.when(i == 0)
    def _():
        se_ref[...] = jnp.zeros_like(se_ref)
        swa_ref[...] = jnp.zeros_like(swa_ref)
        sp_ref[...] = jnp.zeros_like(sp_ref)
        spp_ref[...] = jnp.zeros_like(spp_ref)

    xcat = jnp.concatenate(
        [h0_ref[...], h1_ref[...], h2_ref[...], h3_ref[...]], axis=1)
    hp = jax.nn.relu(_dot(xcat, phiw_ref[...]) + phib_ref[...])
    a = jnp.tanh(_dot(hp, aw_ref[...]) + ab_ref[...])
    g = jax.nn.sigmoid(_dot(hp, bw_ref[...]) + bb_ref[...])
    logit = _dot(a * g, cw_ref[...]) + cb_ref[...]
    e = jnp.exp(logit)
    proj = _dot(hp, vw_ref[...])
    se_ref[...] += jnp.sum(e).reshape(1, 1)
    swa_ref[...] += jnp.sum(e * hp, axis=0, keepdims=True)
    sp_ref[...] += jnp.sum(e * proj, axis=0, keepdims=True)
    spp_ref[...] += jnp.sum(e * proj * proj, axis=0, keepdims=True)

    @pl.when(i == N_ROW_BLOCKS - 1)
    def _():
        se = se_ref[...]
        wavg = swa_ref[...] / se
        mean = sp_ref[...] / se
        var = spp_ref[...] / se - mean * mean
        vp = jnp.log(var + LOG_EPS)
        merged = jnp.concatenate([wavg, vp], axis=1)
        o_ref[...] = _dot(merged, hw_ref[...]) + hb_ref[...]


def _head(hs, p):
    cat = 4 * HID
    nvp = p["var_w"].shape[1]
    body = pl.pallas_call(
        _head_body,
        grid=(N_ROW_BLOCKS,),
        in_specs=[pl.BlockSpec((ROW_BLOCK, HID), lambda i: (i, 0))] * 4 + [
            pl.BlockSpec((cat, cat), lambda i: (0, 0)),
            pl.BlockSpec((1, cat), lambda i: (0, 0)),
            pl.BlockSpec((cat, cat), lambda i: (0, 0)),
            pl.BlockSpec((1, cat), lambda i: (0, 0)),
            pl.BlockSpec((cat, cat), lambda i: (0, 0)),
            pl.BlockSpec((1, cat), lambda i: (0, 0)),
            pl.BlockSpec((cat, 1), lambda i: (0, 0)),
            pl.BlockSpec((1, 1), lambda i: (0, 0)),
            pl.BlockSpec((cat, nvp), lambda i: (0, 0)),
            pl.BlockSpec((cat + nvp, 4), lambda i: (0, 0)),
            pl.BlockSpec((1, 4), lambda i: (0, 0)),
        ],
        out_specs=pl.BlockSpec((1, 4), lambda i: (0, 0)),
        out_shape=jax.ShapeDtypeStruct((1, 4), jnp.float32),
        scratch_shapes=[
            pltpu.VMEM((1, 1), jnp.float32),
            pltpu.VMEM((1, cat), jnp.float32),
            pltpu.VMEM((1, nvp), jnp.float32),
            pltpu.VMEM((1, nvp), jnp.float32),
        ],
        interpret=_INTERPRET,
    )
    return body(hs[0], hs[1], hs[2], hs[3],
                p["phi_w"], p["phi_b"].reshape(1, cat),
                p["attn_a_w"], p["attn_a_b"].reshape(1, cat),
                p["attn_b_w"], p["attn_b_b"].reshape(1, cat),
                p["attn_c_w"], p["attn_c_b"].reshape(1, 1),
                p["var_w"], p["head_w"], p["head_b"].reshape(1, 4))


def kernel(x, edge_index, params):
    p = params
    pad = ((0, 0), (0, _EPS_PAD - _EDGES_PER_SUB))
    src_p = jnp.pad(
        edge_index[0].astype(jnp.int32).reshape(_SC_SUBCORES, _EDGES_PER_SUB),
        pad, constant_values=N_NODES).reshape(_SC_SUBCORES, _NWIN_P, _W)
    dst_p = jnp.pad(
        edge_index[1].astype(jnp.int32).reshape(_SC_SUBCORES, _EDGES_PER_SUB),
        pad, constant_values=0).reshape(_SC_SUBCORES, _NWIN_P, _W)
    h, tab = _fc(x, p["fc_w"], p["fc_b"], p["layers"][0]["t"])
    hs = [h]
    nl = len(p["layers"])
    for i, lp in enumerate(p["layers"]):
        acc = _edge_pass(tab, src_p, dst_p)
        numer = jnp.concatenate([acc[0, :, :64], acc[1, :, :64]], axis=1)
        denom = jnp.concatenate([acc[0, :, 64:], acc[1, :, 64:]], axis=1)
        t_next = p["layers"][i + 1]["t"] if i + 1 < nl else None
        h, tab = _post(h, numer, denom, lp, first=(i == 0), t_next=t_next)
        hs.append(h)
    return _head(hs, p)


# speedup vs baseline: 8.0470x; 1.0512x over previous
import functools

import jax
import jax.numpy as jnp
from jax import lax
from jax.experimental import pallas as pl
from jax.experimental.pallas import tpu as pltpu
from jax.experimental.pallas import tpu_sc as plsc

N_NODES = 10000
N_EDGES = 320000
HID = 128
GEN_EPS = 1e-7
LOG_EPS = 0.01
_PREC = None

ROW_BLOCK = 1000
N_ROW_BLOCKS = N_NODES // ROW_BLOCK

_INTERPRET = False


def _ln(y, g, b):
    mu = jnp.mean(y, axis=-1, keepdims=True)
    var = jnp.mean((y - mu) ** 2, axis=-1, keepdims=True)
    return (y - mu) / jnp.sqrt(var + 1e-5) * g + b


def _dot(a, b):
    return jnp.dot(a, b, preferred_element_type=jnp.float32, precision=_PREC)


def _tab_from_h(h, t):
    m = h + GEN_EPS
    e = jnp.exp(t * m)
    em = e * m
    return jnp.concatenate(
        [em[:, :64], e[:, :64], em[:, 64:], e[:, 64:]], axis=1)


def _fc_body(t_ref, x_ref, w_ref, b_ref, o_ref, tab_ref):
    h = jax.nn.relu(_dot(x_ref[...], w_ref[...]) + b_ref[...])
    o_ref[...] = h
    tab_ref[...] = _tab_from_h(h, t_ref[0, 0])


def _fc(x, w, b, t):
    in_dim = x.shape[1]
    return pl.pallas_call(
        _fc_body,
        grid=(N_ROW_BLOCKS,),
        in_specs=[
            pl.BlockSpec((1, 1), lambda i: (0, 0)),
            pl.BlockSpec((ROW_BLOCK, in_dim), lambda i: (i, 0)),
            pl.BlockSpec((in_dim, HID), lambda i: (0, 0)),
            pl.BlockSpec((1, HID), lambda i: (0, 0)),
        ],
        out_specs=[
            pl.BlockSpec((ROW_BLOCK, HID), lambda i: (i, 0)),
            pl.BlockSpec((ROW_BLOCK, 2 * HID), lambda i: (i, 0)),
        ],
        out_shape=[
            jax.ShapeDtypeStruct((N_NODES, HID), jnp.float32),
            jax.ShapeDtypeStruct((N_NODES, 2 * HID), jnp.float32),
        ],
        interpret=_INTERPRET,
    )(t.reshape(1, 1), x, w, b.reshape(1, HID))


_SC_SUBCORES = 16
_EDGES_PER_SUB = N_EDGES // _SC_SUBCORES
_W = 80
_NWIN = _EDGES_PER_SUB // _W
_RS = 624
_RS_LAST = N_NODES - 15 * _RS


_EPS_PAD = 20480
_NWIN_P = _EPS_PAD // _W
_IG = 8
_NIG = _NWIN_P // _IG
_G = 4


def _sc_edge_body(tab_hbm, src_hbm, dst_hbm, zeros_hbm, out_hbm,
                  gixa, gixb, dsta, dstb, rows_v, acc_sh, isem, gsem, ssem):
    c = lax.axis_index("c")
    s = lax.axis_index("s")
    cv = jnp.broadcast_to(c, (16,)).astype(jnp.int32)
    r0 = s * _RS

    @pl.when(s < 15)
    def _():
        pltpu.sync_copy(zeros_hbm.at[pl.ds(r0, _RS)],
                        acc_sh.at[pl.ds(r0, _RS)])

    @pl.when(s == 15)
    def _():
        pltpu.sync_copy(zeros_hbm.at[pl.ds(15 * _RS, _RS_LAST)],
                        acc_sh.at[pl.ds(15 * _RS, _RS_LAST)])

    def idx_fire(ig, gix, dstw):
        pltpu.async_copy(src_hbm.at[s, pl.ds(ig * _IG, _IG)], gix, isem)
        pltpu.async_copy(dst_hbm.at[s, pl.ds(ig * _IG, _IG)], dstw, isem)

    def idx_wait(gix, dstw):
        pltpu.make_async_copy(src_hbm.at[s, pl.ds(0, _IG)], gix, isem).wait()
        pltpu.make_async_copy(dst_hbm.at[s, pl.ds(0, _IG)], dstw, isem).wait()
        for w in range(_IG):
            for j in range(0, _W, 16):
                gix[w, pl.ds(j, 16)] = gix[w, pl.ds(j, 16)] * 2 + cv

    def process(gix, dstw):
        def g(w, b):
            return pltpu.async_copy(tab_hbm.at[gix.at[w]], rows_v.at[b],
                                    gsem)

        def sc(w, b):
            return pltpu.async_copy(rows_v.at[b], acc_sh.at[dstw.at[w]],
                                    ssem, add=True)

        g0, g1, g2, g3 = g(0, 0), g(1, 1), g(2, 2), g(3, 3)
        g0.wait(); g1.wait()
        s0, s1 = sc(0, 0), sc(1, 1)
        g2.wait(); g3.wait()
        s2, s3 = sc(2, 2), sc(3, 3)
        s0.wait(); s1.wait()
        g4, g5 = g(4, 0), g(5, 1)
        s2.wait(); s3.wait()
        g6, g7 = g(6, 2), g(7, 3)
        g4.wait(); g5.wait()
        s4, s5 = sc(4, 0), sc(5, 1)
        g6.wait(); g7.wait()
        s6, s7 = sc(6, 2), sc(7, 3)
        s4.wait(); s5.wait(); s6.wait(); s7.wait()

    idx_fire(0, gixa, dsta)
    plsc.subcore_barrier()

    @pl.loop(0, _NIG // 2)
    def _grp(u):
        idx_wait(gixa, dsta)
        idx_fire(2 * u + 1, gixb, dstb)
        process(gixa, dsta)
        idx_wait(gixb, dstb)

        @pl.when(u < _NIG // 2 - 1)
        def _():
            idx_fire(2 * u + 2, gixa, dsta)

        process(gixb, dstb)

    plsc.subcore_barrier()

    @pl.when(s < 15)
    def _():
        pltpu.sync_copy(acc_sh.at[pl.ds(r0, _RS)],
                        out_hbm.at[c, pl.ds(r0, _RS)])

    @pl.when(s == 15)
    def _():
        pltpu.sync_copy(acc_sh.at[pl.ds(15 * _RS, _RS_LAST)],
                        out_hbm.at[c, pl.ds(15 * _RS, _RS_LAST)])


def _edge_pass(tab, src_p, dst_p):
    t2 = tab.reshape(2 * N_NODES, HID)
    zeros = jnp.zeros((N_NODES + 8, HID), jnp.float32)
    mesh = plsc.VectorSubcoreMesh(core_axis_name="c", subcore_axis_name="s")
    f = pl.kernel(
        _sc_edge_body,
        out_type=jax.ShapeDtypeStruct((2, N_NODES, HID), jnp.float32),
        mesh=mesh,
        scratch_types=[
            pltpu.VMEM((_IG, _W), jnp.int32),
            pltpu.VMEM((_IG, _W), jnp.int32),
            pltpu.VMEM((_IG, _W), jnp.int32),
            pltpu.VMEM((_IG, _W), jnp.int32),
            pltpu.VMEM((_G, _W, HID), jnp.float32),
            pltpu.VMEM_SHARED((N_NODES + 8, HID), jnp.float32),
            pltpu.SemaphoreType.DMA,
            pltpu.SemaphoreType.DMA,
            pltpu.SemaphoreType.DMA,
        ],
    )
    return f(t2, src_p, dst_p, zeros)


def _post_body(t_ref, h_ref, acc0_ref, acc1_ref, w1_ref, b1_ref, g1_ref,
               bb1_ref, w2_ref, b2_ref, g2_ref, bb2_ref, o_ref, *maybe_tab,
               first, with_tab):
    a0 = acc0_ref[0]
    a1 = acc1_ref[0]
    numer = jnp.concatenate([a0[:, :64], a1[:, :64]], axis=1)
    denom = jnp.concatenate([a0[:, 64:], a1[:, 64:]], axis=1)
    aggr = numer / (denom + 1e-16)
    out = h_ref[...] + aggr
    y = _dot(out, w1_ref[...]) + b1_ref[...]
    y = _ln(y, g1_ref[...], bb1_ref[...])
    y = jax.nn.relu(y)
    y = _dot(y, w2_ref[...]) + b2_ref[...]
    if first:
        h_new = y
    else:
        y = _ln(y, g2_ref[...], bb2_ref[...])
        h_new = h_ref[...] + jax.nn.relu(y)
    o_ref[...] = h_new
    if with_tab:
        maybe_tab[0][...] = _tab_from_h(jax.nn.relu(h_new), t_ref[0, 0])


def _post(h, acc, lp, first, t_next):
    h2 = 2 * HID
    with_tab = t_next is not None
    out_specs = [pl.BlockSpec((ROW_BLOCK, HID), lambda i: (i, 0))]
    out_shape = [jax.ShapeDtypeStruct((N_NODES, HID), jnp.float32)]
    if with_tab:
        out_specs.append(pl.BlockSpec((ROW_BLOCK, 2 * HID), lambda i: (i, 0)))
        out_shape.append(
            jax.ShapeDtypeStruct((N_NODES, 2 * HID), jnp.float32))
    tv = t_next if with_tab else lp["t"]
    res = pl.pallas_call(
        functools.partial(_post_body, first=first, with_tab=with_tab),
        grid=(N_ROW_BLOCKS,),
        in_specs=[
            pl.BlockSpec((1, 1), lambda i: (0, 0)),
            pl.BlockSpec((ROW_BLOCK, HID), lambda i: (i, 0)),
            pl.BlockSpec((1, ROW_BLOCK, HID), lambda i: (0, i, 0)),
            pl.BlockSpec((1, ROW_BLOCK, HID), lambda i: (1, i, 0)),
            pl.BlockSpec((HID, h2), lambda i: (0, 0)),
            pl.BlockSpec((1, h2), lambda i: (0, 0)),
            pl.BlockSpec((1, h2), lambda i: (0, 0)),
            pl.BlockSpec((1, h2), lambda i: (0, 0)),
            pl.BlockSpec((h2, HID), lambda i: (0, 0)),
            pl.BlockSpec((1, HID), lambda i: (0, 0)),
            pl.BlockSpec((1, HID), lambda i: (0, 0)),
            pl.BlockSpec((1, HID), lambda i: (0, 0)),
        ],
        out_specs=out_specs,
        out_shape=out_shape,
        interpret=_INTERPRET,
    )(tv.reshape(1, 1), h, acc, acc,
      lp["w1"], lp["b1"].reshape(1, h2), lp["ln1_g"].reshape(1, h2),
      lp["ln1_b"].reshape(1, h2), lp["w2"], lp["b2"].reshape(1, HID),
      lp["ln_g"].reshape(1, HID), lp["ln_b"].reshape(1, HID))
    return res if with_tab else (res[0], None)


def _head_body(h0_ref, h1_ref, h2_ref, h3_ref, phiw_ref, phib_ref,
               aw_ref, ab_ref, bw_ref, bb_ref, cw_ref, cb_ref, vw_ref,
               hw_ref, hb_ref, o_ref, se_ref, swa_ref, sp_ref, spp_ref):
    i = pl.program_id(0)

    @pl.when(i == 0)
    def _():
        se_ref[...] = jnp.zeros_like(se_ref)
        swa_ref[...] = jnp.zeros_like(swa_ref)
        sp_ref[...] = jnp.zeros_like(sp_ref)
        spp_ref[...] = jnp.zeros_like(spp_ref)

    xcat = jnp.concatenate(
        [h0_ref[...], h1_ref[...], h2_ref[...], h3_ref[...]], axis=1)
    hp = jax.nn.relu(_dot(xcat, phiw_ref[...]) + phib_ref[...])
    a = jnp.tanh(_dot(hp, aw_ref[...]) + ab_ref[...])
    g = jax.nn.sigmoid(_dot(hp, bw_ref[...]) + bb_ref[...])
    logit = _dot(a * g, cw_ref[...]) + cb_ref[...]
    e = jnp.exp(logit)
    proj = _dot(hp, vw_ref[...])
    se_ref[...] += jnp.sum(e).reshape(1, 1)
    swa_ref[...] += jnp.sum(e * hp, axis=0, keepdims=True)
    sp_ref[...] += jnp.sum(e * proj, axis=0, keepdims=True)
    spp_ref[...] += jnp.sum(e * proj * proj, axis=0, keepdims=True)

    @pl.when(i == N_ROW_BLOCKS - 1)
    def _():
        se = se_ref[...]
        wavg = swa_ref[...] / se
        mean = sp_ref[...] / se
        var = spp_ref[...] / se - mean * mean
        vp = jnp.log(var + LOG_EPS)
        merged = jnp.concatenate([wavg, vp], axis=1)
        o_ref[...] = _dot(merged, hw_ref[...]) + hb_ref[...]


def _head(hs, p):
    cat = 4 * HID
    nvp = p["var_w"].shape[1]
    body = pl.pallas_call(
        _head_body,
        grid=(N_ROW_BLOCKS,),
        in_specs=[pl.BlockSpec((ROW_BLOCK, HID), lambda i: (i, 0))] * 4 + [
            pl.BlockSpec((cat, cat), lambda i: (0, 0)),
            pl.BlockSpec((1, cat), lambda i: (0, 0)),
            pl.BlockSpec((cat, cat), lambda i: (0, 0)),
            pl.BlockSpec((1, cat), lambda i: (0, 0)),
            pl.BlockSpec((cat, cat), lambda i: (0, 0)),
            pl.BlockSpec((1, cat), lambda i: (0, 0)),
            pl.BlockSpec((cat, 1), lambda i: (0, 0)),
            pl.BlockSpec((1, 1), lambda i: (0, 0)),
            pl.BlockSpec((cat, nvp), lambda i: (0, 0)),
            pl.BlockSpec((cat + nvp, 4), lambda i: (0, 0)),
            pl.BlockSpec((1, 4), lambda i: (0, 0)),
        ],
        out_specs=pl.BlockSpec((1, 4), lambda i: (0, 0)),
        out_shape=jax.ShapeDtypeStruct((1, 4), jnp.float32),
        scratch_shapes=[
            pltpu.VMEM((1, 1), jnp.float32),
            pltpu.VMEM((1, cat), jnp.float32),
            pltpu.VMEM((1, nvp), jnp.float32),
            pltpu.VMEM((1, nvp), jnp.float32),
        ],
        interpret=_INTERPRET,
    )
    return body(hs[0], hs[1], hs[2], hs[3],
                p["phi_w"], p["phi_b"].reshape(1, cat),
                p["attn_a_w"], p["attn_a_b"].reshape(1, cat),
                p["attn_b_w"], p["attn_b_b"].reshape(1, cat),
                p["attn_c_w"], p["attn_c_b"].reshape(1, 1),
                p["var_w"], p["head_w"], p["head_b"].reshape(1, 4))


def kernel(x, edge_index, params):
    p = params
    pad = ((0, 0), (0, _EPS_PAD - _EDGES_PER_SUB))
    src_p = jnp.pad(
        edge_index[0].astype(jnp.int32).reshape(_SC_SUBCORES, _EDGES_PER_SUB),
        pad, constant_values=0).reshape(_SC_SUBCORES, _NWIN_P, _W)
    dst_p = jnp.pad(
        edge_index[1].astype(jnp.int32).reshape(_SC_SUBCORES, _EDGES_PER_SUB),
        pad, constant_values=N_NODES).reshape(_SC_SUBCORES, _NWIN_P, _W)
    h, tab = _fc(x, p["fc_w"], p["fc_b"], p["layers"][0]["t"])
    hs = [h]
    nl = len(p["layers"])
    for i, lp in enumerate(p["layers"]):
        acc = _edge_pass(tab, src_p, dst_p)
        t_next = p["layers"][i + 1]["t"] if i + 1 < nl else None
        h, tab = _post(h, acc, lp, first=(i == 0), t_next=t_next)
        hs.append(h)
    return _head(hs, p)


# speedup vs baseline: 8.0490x; 1.0003x over previous
import functools

import jax
import jax.numpy as jnp
from jax import lax
from jax.experimental import pallas as pl
from jax.experimental.pallas import tpu as pltpu
from jax.experimental.pallas import tpu_sc as plsc

N_NODES = 10000
N_EDGES = 320000
HID = 128
GEN_EPS = 1e-7
LOG_EPS = 0.01
_PREC = None

ROW_BLOCK = 1000
N_ROW_BLOCKS = N_NODES // ROW_BLOCK

def _ln(y, g, b):
    mu = jnp.mean(y, axis=-1, keepdims=True)
    var = jnp.mean((y - mu) ** 2, axis=-1, keepdims=True)
    return (y - mu) / jnp.sqrt(var + 1e-5) * g + b


def _dot(a, b):
    return jnp.dot(a, b, preferred_element_type=jnp.float32, precision=_PREC)


def _tab_from_h(h, t):
    m = h + GEN_EPS
    e = jnp.exp(t * m)
    em = e * m
    return jnp.concatenate(
        [em[:, :64], e[:, :64], em[:, 64:], e[:, 64:]], axis=1)


def _fc_body(t_ref, x_ref, w_ref, b_ref, o_ref, tab_ref):
    h = jax.nn.relu(_dot(x_ref[...], w_ref[...]) + b_ref[...])
    o_ref[...] = h
    tab_ref[...] = _tab_from_h(h, t_ref[0, 0])


def _fc(x, w, b, t):
    in_dim = x.shape[1]
    return pl.pallas_call(
        _fc_body,
        grid=(N_ROW_BLOCKS,),
        in_specs=[
            pl.BlockSpec((1, 1), lambda i: (0, 0)),
            pl.BlockSpec((ROW_BLOCK, in_dim), lambda i: (i, 0)),
            pl.BlockSpec((in_dim, HID), lambda i: (0, 0)),
            pl.BlockSpec((1, HID), lambda i: (0, 0)),
        ],
        out_specs=[
            pl.BlockSpec((ROW_BLOCK, HID), lambda i: (i, 0)),
            pl.BlockSpec((ROW_BLOCK, 2 * HID), lambda i: (i, 0)),
        ],
        out_shape=[
            jax.ShapeDtypeStruct((N_NODES, HID), jnp.float32),
            jax.ShapeDtypeStruct((N_NODES, 2 * HID), jnp.float32),
        ],
    )(t.reshape(1, 1), x, w, b.reshape(1, HID))


_SC_SUBCORES = 16
_EDGES_PER_SUB = N_EDGES // _SC_SUBCORES
_W = 80
_NWIN = _EDGES_PER_SUB // _W
_RS = 624
_RS_LAST = N_NODES - 15 * _RS


_EPS_PAD = 20480
_NWIN_P = _EPS_PAD // _W
_IG = 8
_NIG = _NWIN_P // _IG
_G = 4


def _sc_edge_body(tab_hbm, src_hbm, dst_hbm, zeros_hbm, out_hbm,
                  gixa, gixb, dsta, dstb, rows_v, acc_sh, isem, gsem, ssem):
    c = lax.axis_index("c")
    s = lax.axis_index("s")
    cv = jnp.broadcast_to(c, (16,)).astype(jnp.int32)
    r0 = s * _RS

    @pl.when(s < 15)
    def _():
        pltpu.sync_copy(zeros_hbm.at[pl.ds(r0, _RS)],
                        acc_sh.at[pl.ds(r0, _RS)])

    @pl.when(s == 15)
    def _():
        pltpu.sync_copy(zeros_hbm.at[pl.ds(15 * _RS, _RS_LAST)],
                        acc_sh.at[pl.ds(15 * _RS, _RS_LAST)])

    def idx_fire(ig, gix, dstw):
        pltpu.async_copy(src_hbm.at[s, pl.ds(ig * _IG, _IG)], gix, isem)
        pltpu.async_copy(dst_hbm.at[s, pl.ds(ig * _IG, _IG)], dstw, isem)

    def idx_wait(gix, dstw):
        pltpu.make_async_copy(src_hbm.at[s, pl.ds(0, _IG)], gix, isem).wait()
        pltpu.make_async_copy(dst_hbm.at[s, pl.ds(0, _IG)], dstw, isem).wait()
        for w in range(_IG):
            for j in range(0, _W, 16):
                gix[w, pl.ds(j, 16)] = gix[w, pl.ds(j, 16)] * 2 + cv

    def process(gix, dstw):
        def g(w, b):
            return pltpu.async_copy(tab_hbm.at[gix.at[w]], rows_v.at[b],
                                    gsem)

        def sc(w, b):
            return pltpu.async_copy(rows_v.at[b], acc_sh.at[dstw.at[w]],
                                    ssem, add=True)

        g0, g1, g2, g3 = g(0, 0), g(1, 1), g(2, 2), g(3, 3)
        g0.wait(); g1.wait()
        s0, s1 = sc(0, 0), sc(1, 1)
        g2.wait(); g3.wait()
        s2, s3 = sc(2, 2), sc(3, 3)
        s0.wait(); s1.wait()
        g4, g5 = g(4, 0), g(5, 1)
        s2.wait(); s3.wait()
        g6, g7 = g(6, 2), g(7, 3)
        g4.wait(); g5.wait()
        s4, s5 = sc(4, 0), sc(5, 1)
        g6.wait(); g7.wait()
        s6, s7 = sc(6, 2), sc(7, 3)
        s4.wait(); s5.wait(); s6.wait(); s7.wait()

    idx_fire(0, gixa, dsta)
    plsc.subcore_barrier()

    @pl.loop(0, _NIG // 2)
    def _grp(u):
        idx_wait(gixa, dsta)
        idx_fire(2 * u + 1, gixb, dstb)
        process(gixa, dsta)
        idx_wait(gixb, dstb)

        @pl.when(u < _NIG // 2 - 1)
        def _():
            idx_fire(2 * u + 2, gixa, dsta)

        process(gixb, dstb)

    plsc.subcore_barrier()

    @pl.when(s < 15)
    def _():
        pltpu.sync_copy(acc_sh.at[pl.ds(r0, _RS)],
                        out_hbm.at[c, pl.ds(r0, _RS)])

    @pl.when(s == 15)
    def _():
        pltpu.sync_copy(acc_sh.at[pl.ds(15 * _RS, _RS_LAST)],
                        out_hbm.at[c, pl.ds(15 * _RS, _RS_LAST)])


def _edge_pass(tab, src_p, dst_p):
    t2 = tab.reshape(2 * N_NODES, HID)
    zeros = jnp.zeros((N_NODES + 8, HID), jnp.float32)
    mesh = plsc.VectorSubcoreMesh(core_axis_name="c", subcore_axis_name="s")
    f = pl.kernel(
        _sc_edge_body,
        out_type=jax.ShapeDtypeStruct((2, N_NODES, HID), jnp.float32),
        mesh=mesh,
        scratch_types=[
            pltpu.VMEM((_IG, _W), jnp.int32),
            pltpu.VMEM((_IG, _W), jnp.int32),
            pltpu.VMEM((_IG, _W), jnp.int32),
            pltpu.VMEM((_IG, _W), jnp.int32),
            pltpu.VMEM((_G, _W, HID), jnp.float32),
            pltpu.VMEM_SHARED((N_NODES + 8, HID), jnp.float32),
            pltpu.SemaphoreType.DMA,
            pltpu.SemaphoreType.DMA,
            pltpu.SemaphoreType.DMA,
        ],
    )
    return f(t2, src_p, dst_p, zeros)


def _post_body(t_ref, h_ref, acc0_ref, acc1_ref, w1_ref, b1_ref, g1_ref,
               bb1_ref, w2_ref, b2_ref, g2_ref, bb2_ref, o_ref, *maybe_tab,
               first, with_tab):
    a0 = acc0_ref[0]
    a1 = acc1_ref[0]
    numer = jnp.concatenate([a0[:, :64], a1[:, :64]], axis=1)
    denom = jnp.concatenate([a0[:, 64:], a1[:, 64:]], axis=1)
    aggr = numer / (denom + 1e-16)
    out = h_ref[...] + aggr
    y = _dot(out, w1_ref[...]) + b1_ref[...]
    y = _ln(y, g1_ref[...], bb1_ref[...])
    y = jax.nn.relu(y)
    y = _dot(y, w2_ref[...]) + b2_ref[...]
    if first:
        h_new = y
    else:
        y = _ln(y, g2_ref[...], bb2_ref[...])
        h_new = h_ref[...] + jax.nn.relu(y)
    o_ref[...] = h_new
    if with_tab:
        maybe_tab[0][...] = _tab_from_h(jax.nn.relu(h_new), t_ref[0, 0])


def _post(h, acc, lp, first, t_next):
    h2 = 2 * HID
    with_tab = t_next is not None
    out_specs = [pl.BlockSpec((ROW_BLOCK, HID), lambda i: (i, 0))]
    out_shape = [jax.ShapeDtypeStruct((N_NODES, HID), jnp.float32)]
    if with_tab:
        out_specs.append(pl.BlockSpec((ROW_BLOCK, 2 * HID), lambda i: (i, 0)))
        out_shape.append(
            jax.ShapeDtypeStruct((N_NODES, 2 * HID), jnp.float32))
    tv = t_next if with_tab else lp["t"]
    res = pl.pallas_call(
        functools.partial(_post_body, first=first, with_tab=with_tab),
        grid=(N_ROW_BLOCKS,),
        in_specs=[
            pl.BlockSpec((1, 1), lambda i: (0, 0)),
            pl.BlockSpec((ROW_BLOCK, HID), lambda i: (i, 0)),
            pl.BlockSpec((1, ROW_BLOCK, HID), lambda i: (0, i, 0)),
            pl.BlockSpec((1, ROW_BLOCK, HID), lambda i: (1, i, 0)),
            pl.BlockSpec((HID, h2), lambda i: (0, 0)),
            pl.BlockSpec((1, h2), lambda i: (0, 0)),
            pl.BlockSpec((1, h2), lambda i: (0, 0)),
            pl.BlockSpec((1, h2), lambda i: (0, 0)),
            pl.BlockSpec((h2, HID), lambda i: (0, 0)),
            pl.BlockSpec((1, HID), lambda i: (0, 0)),
            pl.BlockSpec((1, HID), lambda i: (0, 0)),
            pl.BlockSpec((1, HID), lambda i: (0, 0)),
        ],
        out_specs=out_specs,
        out_shape=out_shape,
    )(tv.reshape(1, 1), h, acc, acc,
      lp["w1"], lp["b1"].reshape(1, h2), lp["ln1_g"].reshape(1, h2),
      lp["ln1_b"].reshape(1, h2), lp["w2"], lp["b2"].reshape(1, HID),
      lp["ln_g"].reshape(1, HID), lp["ln_b"].reshape(1, HID))
    return res if with_tab else (res[0], None)


def _head_body(h0_ref, h1_ref, h2_ref, h3_ref, phiw_ref, phib_ref,
               aw_ref, ab_ref, bw_ref, bb_ref, cw_ref, cb_ref, vw_ref,
               hw_ref, hb_ref, o_ref, se_ref, swa_ref, sp_ref, spp_ref):
    i = pl.program_id(0)

    @pl.when(i == 0)
    def _():
        se_ref[...] = jnp.zeros_like(se_ref)
        swa_ref[...] = jnp.zeros_like(swa_ref)
        sp_ref[...] = jnp.zeros_like(sp_ref)
        spp_ref[...] = jnp.zeros_like(spp_ref)

    xcat = jnp.concatenate(
        [h0_ref[...], h1_ref[...], h2_ref[...], h3_ref[...]], axis=1)
    hp = jax.nn.relu(_dot(xcat, phiw_ref[...]) + phib_ref[...])
    a = jnp.tanh(_dot(hp, aw_ref[...]) + ab_ref[...])
    g = jax.nn.sigmoid(_dot(hp, bw_ref[...]) + bb_ref[...])
    logit = _dot(a * g, cw_ref[...]) + cb_ref[...]
    e = jnp.exp(logit)
    proj = _dot(hp, vw_ref[...])
    se_ref[...] += jnp.sum(e).reshape(1, 1)
    swa_ref[...] += jnp.sum(e * hp, axis=0, keepdims=True)
    sp_ref[...] += jnp.sum(e * proj, axis=0, keepdims=True)
    spp_ref[...] += jnp.sum(e * proj * proj, axis=0, keepdims=True)

    @pl.when(i == N_ROW_BLOCKS - 1)
    def _():
        se = se_ref[...]
        wavg = swa_ref[...] / se
        mean = sp_ref[...] / se
        var = spp_ref[...] / se - mean * mean
        vp = jnp.log(var + LOG_EPS)
        merged = jnp.concatenate([wavg, vp], axis=1)
        o_ref[...] = _dot(merged, hw_ref[...]) + hb_ref[...]


def _head(hs, p):
    cat = 4 * HID
    nvp = p["var_w"].shape[1]
    body = pl.pallas_call(
        _head_body,
        grid=(N_ROW_BLOCKS,),
        in_specs=[pl.BlockSpec((ROW_BLOCK, HID), lambda i: (i, 0))] * 4 + [
            pl.BlockSpec((cat, cat), lambda i: (0, 0)),
            pl.BlockSpec((1, cat), lambda i: (0, 0)),
            pl.BlockSpec((cat, cat), lambda i: (0, 0)),
            pl.BlockSpec((1, cat), lambda i: (0, 0)),
            pl.BlockSpec((cat, cat), lambda i: (0, 0)),
            pl.BlockSpec((1, cat), lambda i: (0, 0)),
            pl.BlockSpec((cat, 1), lambda i: (0, 0)),
            pl.BlockSpec((1, 1), lambda i: (0, 0)),
            pl.BlockSpec((cat, nvp), lambda i: (0, 0)),
            pl.BlockSpec((cat + nvp, 4), lambda i: (0, 0)),
            pl.BlockSpec((1, 4), lambda i: (0, 0)),
        ],
        out_specs=pl.BlockSpec((1, 4), lambda i: (0, 0)),
        out_shape=jax.ShapeDtypeStruct((1, 4), jnp.float32),
        scratch_shapes=[
            pltpu.VMEM((1, 1), jnp.float32),
            pltpu.VMEM((1, cat), jnp.float32),
            pltpu.VMEM((1, nvp), jnp.float32),
            pltpu.VMEM((1, nvp), jnp.float32),
        ],
    )
    return body(hs[0], hs[1], hs[2], hs[3],
                p["phi_w"], p["phi_b"].reshape(1, cat),
                p["attn_a_w"], p["attn_a_b"].reshape(1, cat),
                p["attn_b_w"], p["attn_b_b"].reshape(1, cat),
                p["attn_c_w"], p["attn_c_b"].reshape(1, 1),
                p["var_w"], p["head_w"], p["head_b"].reshape(1, 4))


def kernel(x, edge_index, params):
    p = params
    pad = ((0, 0), (0, _EPS_PAD - _EDGES_PER_SUB))
    src_p = jnp.pad(
        edge_index[0].astype(jnp.int32).reshape(_SC_SUBCORES, _EDGES_PER_SUB),
        pad, constant_values=0).reshape(_SC_SUBCORES, _NWIN_P, _W)
    dst_p = jnp.pad(
        edge_index[1].astype(jnp.int32).reshape(_SC_SUBCORES, _EDGES_PER_SUB),
        pad, constant_values=N_NODES).reshape(_SC_SUBCORES, _NWIN_P, _W)
    h, tab = _fc(x, p["fc_w"], p["fc_b"], p["layers"][0]["t"])
    hs = [h]
    nl = len(p["layers"])
    for i, lp in enumerate(p["layers"]):
        acc = _edge_pass(tab, src_p, dst_p)
        t_next = p["layers"][i + 1]["t"] if i + 1 < nl else None
        h, tab = _post(h, acc, lp, first=(i == 0), t_next=t_next)
        hs.append(h)
    return _head(hs, p)
